# baseline (speedup 1.0000x reference)
import functools

import jax
import jax.numpy as jnp
from jax import lax
from jax.experimental import pallas as pl
from jax.experimental.pallas import tpu as pltpu
from jax.experimental.pallas import tpu_sc as plsc

N_ERA = 50000
N_H = 10000
E = 160000
IN_CH = 128
HID = 128

_INTERPRET = False


def _ln(x, g, b):
    mu = jnp.mean(x, axis=-1, keepdims=True)
    var = jnp.mean((x - mu) ** 2, axis=-1, keepdims=True)
    return (x - mu) * jax.lax.rsqrt(var + 1e-5) * g + b


def _silu(x):
    return x * jax.nn.sigmoid(x)


def _dot(a, b):
    return jnp.dot(a, b, preferred_element_type=jnp.float32)


def _edge_embed_body(attr, w1, b1, w2, b2, g, bln, e_out):
    h = _silu(_dot(attr[...], w1[...]) + b1[...])
    e_out[...] = _ln(_dot(h, w2[...]) + b2[...], g[...], bln[...])


def _edge_embed(attr, p, rb=2000):
    n = attr.shape[0]
    grid = (n // rb,)
    full = lambda shp: pl.BlockSpec(shp, lambda i: (0, 0))
    return pl.pallas_call(
        _edge_embed_body,
        grid=grid,
        in_specs=[
            pl.BlockSpec((rb, 4), lambda i: (i, 0)),
            full((4, HID)), full((1, HID)), full((HID, HID)), full((1, HID)),
            full((1, HID)), full((1, HID)),
        ],
        out_specs=pl.BlockSpec((rb, HID), lambda i: (i, 0)),
        out_shape=jax.ShapeDtypeStruct((n, HID), jnp.float32),
        interpret=_INTERPRET,
    )(attr, p['w1'], p['b1'].reshape(1, -1), p['w2'], p['b2'].reshape(1, -1),
      p['g'].reshape(1, -1), p['bln'].reshape(1, -1))


def _src_embed_body(x, ll, w1x, w1l, b1, w2, b2, g, bln, a_w, bdec_w,
                    xs_out, ps_out, pd_out):
    h = _silu(_dot(x[...], w1x[...]) + _dot(ll[...], w1l[...]) + b1[...])
    xs = _ln(_dot(h, w2[...]) + b2[...], g[...], bln[...])
    xs_out[...] = xs
    ps_out[...] = _dot(xs, a_w[...])
    pd_out[...] = _dot(xs, bdec_w[...])


def _src_embed(x, ll, p, a_w, bdec_w, rb=2000):
    n = x.shape[0]
    grid = (n // rb,)
    full = lambda shp: pl.BlockSpec(shp, lambda i: (0, 0))
    return pl.pallas_call(
        _src_embed_body,
        grid=grid,
        in_specs=[
            pl.BlockSpec((rb, IN_CH), lambda i: (i, 0)),
            pl.BlockSpec((rb, 4), lambda i: (i, 0)),
            full((IN_CH, HID)), full((4, HID)), full((1, HID)),
            full((HID, HID)), full((1, HID)), full((1, HID)), full((1, HID)),
            full((HID, HID)), full((HID, HID)),
        ],
        out_specs=[pl.BlockSpec((rb, HID), lambda i: (i, 0))] * 3,
        out_shape=[jax.ShapeDtypeStruct((n, HID), jnp.float32)] * 3,
        interpret=_INTERPRET,
    )(x, ll, p['w1'][:IN_CH], p['w1'][IN_CH:], p['b1'].reshape(1, -1),
      p['w2'], p['b2'].reshape(1, -1), p['g'].reshape(1, -1),
      p['bln'].reshape(1, -1), a_w, bdec_w)


def _dst_embed_body(ll, w1, b1, w2, b2, g, bln, benc_w, xd_out, pd_out):
    h = _silu(_dot(ll[...], w1[...]) + b1[...])
    xd = _ln(_dot(h, w2[...]) + b2[...], g[...], bln[...])
    xd_out[...] = xd
    pd_out[...] = _dot(xd, benc_w[...])


def _dst_embed(ll, p, benc_w, rb=2000):
    n = ll.shape[0]
    grid = (n // rb,)
    full = lambda shp: pl.BlockSpec(shp, lambda i: (0, 0))
    return pl.pallas_call(
        _dst_embed_body,
        grid=grid,
        in_specs=[
            pl.BlockSpec((rb, 4), lambda i: (i, 0)),
            full((4, HID)), full((1, HID)), full((HID, HID)), full((1, HID)),
            full((1, HID)), full((1, HID)), full((HID, HID)),
        ],
        out_specs=[pl.BlockSpec((rb, HID), lambda i: (i, 0))] * 2,
        out_shape=[jax.ShapeDtypeStruct((n, HID), jnp.float32)] * 2,
        interpret=_INTERPRET,
    )(ll, p['w1'], p['b1'].reshape(1, -1), p['w2'], p['b2'].reshape(1, -1),
      p['g'].reshape(1, -1), p['bln'].reshape(1, -1), benc_w)


def _edge_msg_body(sgd, e, c_w, b1, w2, b2, g, bln, m_out):
    h = _silu(sgd[...] + _dot(e[...], c_w[...]) + b1[...])
    m_out[...] = _ln(_dot(h, w2[...]) + b2[...], g[...], bln[...]) + e[...]


def _edge_msg(sgd, e, p, rb=2000):
    n = sgd.shape[0]
    grid = (n // rb,)
    full = lambda shp: pl.BlockSpec(shp, lambda i: (0, 0))
    return pl.pallas_call(
        _edge_msg_body,
        grid=grid,
        in_specs=[
            pl.BlockSpec((rb, HID), lambda i: (i, 0)),
            pl.BlockSpec((rb, HID), lambda i: (i, 0)),
            full((HID, HID)), full((1, HID)), full((HID, HID)), full((1, HID)),
            full((1, HID)), full((1, HID)),
        ],
        out_specs=pl.BlockSpec((rb, HID), lambda i: (i, 0)),
        out_shape=jax.ShapeDtypeStruct((n, HID), jnp.float32),
        interpret=_INTERPRET,
    )(sgd, e, p['w1'][2 * HID:], p['b1'].reshape(1, -1), p['w2'],
      p['b2'].reshape(1, -1), p['g'].reshape(1, -1), p['bln'].reshape(1, -1))


def _node_update_body(project, xd, agg, v1a, v1b, b1, w2, b2, g, bln, pw, pb,
                      out0, out1=None):
    h = _silu(_dot(xd[...], v1a[...]) + _dot(agg[...], v1b[...]) + b1[...])
    xn = xd[...] + _ln(_dot(h, w2[...]) + b2[...], g[...], bln[...])
    if project:
        out0[...] = _dot(xn, pw[...]) + pb[...]
    else:
        out0[...] = xn
        out1[...] = _dot(xn, pw[...]) + pb[...]


def _node_update(xd, agg, p, pw, pb, project, rb=2000):
    n = xd.shape[0]
    grid = (n // rb,)
    full = lambda shp: pl.BlockSpec(shp, lambda i: (0, 0))
    pout = pw.shape[1]
    if project:
        out_specs = pl.BlockSpec((rb, pout), lambda i: (i, 0))
        out_shape = jax.ShapeDtypeStruct((n, pout), jnp.float32)
    else:
        out_specs = [pl.BlockSpec((rb, HID), lambda i: (i, 0)),
                     pl.BlockSpec((rb, pout), lambda i: (i, 0))]
        out_shape = [jax.ShapeDtypeStruct((n, HID), jnp.float32),
                     jax.ShapeDtypeStruct((n, pout), jnp.float32)]
    return pl.pallas_call(
        functools.partial(_node_update_body, project),
        grid=grid,
        in_specs=[
            pl.BlockSpec((rb, HID), lambda i: (i, 0)),
            pl.BlockSpec((rb, HID), lambda i: (i, 0)),
            full((HID, HID)), full((HID, HID)), full((1, HID)),
            full((HID, HID)), full((1, HID)), full((1, HID)), full((1, HID)),
            full((HID, pout)), full((1, pout)),
        ],
        out_specs=out_specs,
        out_shape=out_shape,
        interpret=_INTERPRET,
    )(xd, agg, p['w1'][:HID], p['w1'][HID:], p['b1'].reshape(1, -1),
      p['w2'], p['b2'].reshape(1, -1), p['g'].reshape(1, -1),
      p['bln'].reshape(1, -1), pw, pb.reshape(1, -1))


_NC = 2
_NS = 16
_NW = _NC * _NS


def _gather_add(ps, pd, src_idx, dst_idx):
    n = src_idx.shape[0]
    ch = n // _NW
    K = 200
    nch = ch // K
    assert ch * _NW == n and nch * K == ch and K % 8 == 0 and nch % 2 == 1

    mesh = plsc.VectorSubcoreMesh(core_axis_name="c", subcore_axis_name="s")
    vm = lambda *s: pltpu.VMEM(s, jnp.float32)

    @functools.partial(
        pl.kernel, mesh=mesh,
        out_type=jax.ShapeDtypeStruct((n, HID), jnp.float32),
        scratch_types=[
            pltpu.VMEM((K,), jnp.int32), pltpu.VMEM((K,), jnp.int32),
            pltpu.VMEM((K,), jnp.int32), pltpu.VMEM((K,), jnp.int32),
            vm(K, HID), vm(K, HID), vm(K, HID), vm(K, HID),
            pltpu.SemaphoreType.DMA, pltpu.SemaphoreType.DMA,
            pltpu.SemaphoreType.DMA, pltpu.SemaphoreType.DMA,
        ],
    )
    def k(ps_hbm, pd_hbm, si_hbm, di_hbm, out_hbm, si0, si1, di0, di1,
          ra0, rb0, ra1, rb1, sa0, sb0, sa1, sb1):
        wid = lax.axis_index("s") * _NC + lax.axis_index("c")
        base0 = wid * ch
        sis, dis = (si0, si1), (di0, di1)
        ras, rbs = (ra0, ra1), (rb0, rb1)
        sas, sbs = (sa0, sa1), (sb0, sb1)

        def issue(c, b):
            base = base0 + c * K
            pltpu.sync_copy(si_hbm.at[pl.ds(base, K)], sis[b])
            pltpu.sync_copy(di_hbm.at[pl.ds(base, K)], dis[b])
            pltpu.async_copy(ps_hbm.at[sis[b]], ras[b], sas[b])
            pltpu.async_copy(pd_hbm.at[dis[b]], rbs[b], sbs[b])

        def finish(c, b):
            ra, rb = ras[b], rbs[b]
            pltpu.make_async_copy(ps_hbm.at[sis[b]], ra, sas[b]).wait()
            pltpu.make_async_copy(pd_hbm.at[dis[b]], rb, sbs[b]).wait()

            def row(r, c2):
                for j in range(HID // 16):
                    sl = pl.ds(j * 16, 16)
                    rb[r, sl] = ra[r, sl] + rb[r, sl]
                return c2
            lax.fori_loop(0, K, row, 0)
            pltpu.sync_copy(rb, out_hbm.at[pl.ds(base0 + c * K, K)])

        issue(0, 0)
        issue(1, 1)

        def pair(i, carry):
            for b in range(2):
                c = 2 * i + b
                finish(c, b)

                @pl.when(c + 2 < nch)
                def _next():
                    issue(c + 2, b)
            return carry
        lax.fori_loop(0, nch // 2, pair, 0)
        finish(nch - 1, (nch - 1) % 2)

    return k(ps, pd, src_idx, dst_idx)


def _seg_sum_kernel(n_seg, e_total):
    tile_rows = -(-n_seg // (_NW * 8)) * 8
    passes = -(-tile_rows // 784)
    sub = -(-tile_rows // (passes * 8)) * 8
    out_rows = _NW * sub * passes
    DUMP = sub
    IDC = 640
    nidc = e_total // IDC
    assert nidc * IDC == e_total
    EPAD = (-(-e_total // 1024) + 2) * 1024
    mesh = plsc.VectorSubcoreMesh(core_axis_name="c", subcore_axis_name="s")

    @functools.partial(
        pl.kernel, mesh=mesh,
        out_type=[jax.ShapeDtypeStruct((out_rows, HID), jnp.float32),
                  jax.ShapeDtypeStruct((_NW, EPAD), jnp.int32),
                  jax.ShapeDtypeStruct((_NW, EPAD), jnp.int32)],
        scratch_types=[
            pltpu.VMEM((sub + 1, HID), jnp.float32),
            pltpu.VMEM((IDC,), jnp.int32),
            pltpu.VMEM((2048,), jnp.int32),
            pltpu.VMEM((2048,), jnp.int32),
            pltpu.VMEM((1024,), jnp.int32),
            pltpu.VMEM((128, HID), jnp.float32),
            pltpu.SMEM((1024,), jnp.int32),
            pltpu.SemaphoreType.DMA,
        ],
    )
    def k(m_hbm, di_hbm, agg, bin_d, bin_e, acc, idb, cb_d, cb_e,
          blk_e, rows, sm_d, sem):
        t = lax.axis_index("s") * _NC + lax.axis_index("c")
        iota = lax.iota(jnp.int32, 16)

        for p in range(passes):
            lo = t * sub * passes + p * sub
            hi = lo + sub
            def zrow(r, c_):
                for j in range(HID // 16):
                    acc[r, pl.ds(16 * j, 16)] = jnp.zeros((16,), jnp.float32)
                return c_
            lax.fori_loop(0, sub + 1, zrow, 0)

            def chunk(ci, carry):
                pos, nblk = carry
                pltpu.sync_copy(di_hbm.at[pl.ds(ci * IDC, IDC)], idb)

                def vreg(j, pos2):
                    d = idb[pl.ds(j * 16, 16)]
                    msk = (d >= lo) & (d < hi)
                    eidv = iota + (ci * IDC + j * 16)
                    inc = jnp.cumsum(msk.astype(jnp.int32))
                    idxv = pos2 + inc - 1
                    plsc.store_scatter(cb_d, [idxv], d, mask=msk)
                    plsc.store_scatter(cb_e, [idxv], eidv, mask=msk)
                    return pos2 + jnp.max(inc)
                pos = lax.fori_loop(0, IDC // 16, vreg, pos)

                @pl.when(pos >= 1024)
                def _drain():
                    pltpu.sync_copy(cb_d.at[pl.ds(0, 1024)],
                                    bin_d.at[t, pl.ds(nblk * 1024, 1024)])
                    pltpu.sync_copy(cb_e.at[pl.ds(0, 1024)],
                                    bin_e.at[t, pl.ds(nblk * 1024, 1024)])
                    for v in range(64):
                        s_, dsl = pl.ds(1024 + 16 * v, 16), pl.ds(16 * v, 16)
                        cb_d[dsl] = cb_d[s_]
                        cb_e[dsl] = cb_e[s_]
                drained = (pos >= 1024).astype(jnp.int32)
                return pos - 1024 * drained, nblk + drained
            pos, nblk = lax.fori_loop(0, nidc, chunk, (0, 0))

            pltpu.sync_copy(cb_d.at[pl.ds(0, 1024)],
                            bin_d.at[t, pl.ds(nblk * 1024, 1024)])
            pltpu.sync_copy(cb_e.at[pl.ds(0, 1024)],
                            bin_e.at[t, pl.ds(nblk * 1024, 1024)])
            total = nblk * 1024 + pos
            nblk_tot = nblk + (pos > 0).astype(jnp.int32)

            def blk(b, c_):
                pltpu.sync_copy(bin_d.at[t, pl.ds(b * 1024, 1024)], sm_d)
                pltpu.sync_copy(bin_e.at[t, pl.ds(b * 1024, 1024)], blk_e)
                for v in range(64):
                    sl = pl.ds(16 * v, 16)
                    gi = iota + (b * 1024 + 16 * v)
                    blk_e[sl] = jnp.where(gi < total, blk_e[sl], 0)

                def subblk(s_, c2):
                    pltpu.async_copy(
                        m_hbm.at[blk_e.at[pl.ds(128 * s_, 128)]], rows,
                        sem).wait()

                    def row(r, c3):
                        d = sm_d[128 * s_ + r]
                        gi = b * 1024 + 128 * s_ + r
                        ok = (d >= lo) & (d < hi) & (gi < total)
                        off = jnp.where(ok, d - lo, DUMP)
                        for j in range(HID // 16):
                            sl = pl.ds(16 * j, 16)
                            plsc.addupdate(acc.at[off, sl], rows[r, sl])
                        return c3
                    return lax.fori_loop(0, 128, row, c2)
                return lax.fori_loop(0, 8, subblk, c_)
            lax.fori_loop(0, nblk_tot, blk, 0)

            pltpu.sync_copy(acc.at[pl.ds(0, sub)], agg.at[pl.ds(lo, sub)])
    return k


@functools.lru_cache(maxsize=None)
def _seg_sum_fn(n_seg, e_total):
    return _seg_sum_kernel(n_seg, e_total)


def _seg_sum(m, dst_idx, n_seg):
    return jax.ops.segment_sum(m, dst_idx, num_segments=n_seg)


def kernel(x, params, era_latlons, h_latlons, e2h_edge_attr, h2e_edge_attr,
           e2h_edge_index, h2e_edge_index):
    enc, dec = params['enc'], params['dec']
    bs = x.shape[0]
    x_flat = x.reshape(bs * N_ERA, IN_CH)

    a_enc = enc['blk0_edge']['w1'][:HID]
    b_enc = enc['blk0_edge']['w1'][HID:2 * HID]
    a_dec = dec['blk0_edge']['w1'][:HID]
    b_dec = dec['blk0_edge']['w1'][HID:2 * HID]

    e1 = _edge_embed(e2h_edge_attr, enc['emb_edges'])
    xs, ps1, pd2 = _src_embed(x_flat, era_latlons, enc['emb_src'],
                              a_enc, b_dec)
    xd, pd1 = _dst_embed(h_latlons, enc['emb_dst'], b_enc)

    sgd1 = _gather_add(ps1, pd1, e2h_edge_index[0], e2h_edge_index[1])
    m1 = _edge_msg(sgd1, e1, enc['blk0_edge'])
    agg1 = _seg_sum(m1, e2h_edge_index[1], N_H)
    xlat, ps2 = _node_update(xd, agg1, enc['blk0_node'], a_dec,
                             jnp.zeros((HID,), jnp.float32), project=False)

    e2 = _edge_embed(h2e_edge_attr, dec['emb_edges'])
    sgd2 = _gather_add(ps2, pd2, h2e_edge_index[0], h2e_edge_index[1])
    m2 = _edge_msg(sgd2, e2, dec['blk0_edge'])
    agg2 = _seg_sum(m2, h2e_edge_index[1], N_ERA)
    out = _node_update(xs, agg2, dec['blk0_node'], dec['out_w'],
                       dec['out_b'], project=True)
    return out.reshape(bs, N_ERA, IN_CH)


# speedup vs baseline: 1.0263x; 1.0263x over previous
import functools

import jax
import jax.numpy as jnp
from jax import lax
from jax.experimental import pallas as pl
from jax.experimental.pallas import tpu as pltpu
from jax.experimental.pallas import tpu_sc as plsc

N_ERA = 50000
N_H = 10000
E = 160000
IN_CH = 128
HID = 128

_INTERPRET = False


def _ln(x, g, b):
    mu = jnp.mean(x, axis=-1, keepdims=True)
    var = jnp.mean((x - mu) ** 2, axis=-1, keepdims=True)
    return (x - mu) * jax.lax.rsqrt(var + 1e-5) * g + b


def _silu(x):
    return x * jax.nn.sigmoid(x)


def _dot(a, b):
    return jnp.dot(a, b, preferred_element_type=jnp.float32)


def _edge_embed_body(attr, w1, b1, w2, b2, g, bln, e_out):
    h = _silu(_dot(attr[...], w1[...]) + b1[...])
    e_out[...] = _ln(_dot(h, w2[...]) + b2[...], g[...], bln[...])


def _edge_embed(attr, p, rb=2000):
    n = attr.shape[0]
    grid = (n // rb,)
    full = lambda shp: pl.BlockSpec(shp, lambda i: (0, 0))
    return pl.pallas_call(
        _edge_embed_body,
        grid=grid,
        in_specs=[
            pl.BlockSpec((rb, 4), lambda i: (i, 0)),
            full((4, HID)), full((1, HID)), full((HID, HID)), full((1, HID)),
            full((1, HID)), full((1, HID)),
        ],
        out_specs=pl.BlockSpec((rb, HID), lambda i: (i, 0)),
        out_shape=jax.ShapeDtypeStruct((n, HID), jnp.float32),
        interpret=_INTERPRET,
    )(attr, p['w1'], p['b1'].reshape(1, -1), p['w2'], p['b2'].reshape(1, -1),
      p['g'].reshape(1, -1), p['bln'].reshape(1, -1))


def _src_embed_body(x, ll, w1x, w1l, b1, w2, b2, g, bln, a_w, bdec_w,
                    xs_out, ps_out, pd_out):
    h = _silu(_dot(x[...], w1x[...]) + _dot(ll[...], w1l[...]) + b1[...])
    xs = _ln(_dot(h, w2[...]) + b2[...], g[...], bln[...])
    xs_out[...] = xs
    ps_out[...] = _dot(xs, a_w[...])
    pd_out[...] = _dot(xs, bdec_w[...])


def _src_embed(x, ll, p, a_w, bdec_w, rb=2000):
    n = x.shape[0]
    grid = (n // rb,)
    full = lambda shp: pl.BlockSpec(shp, lambda i: (0, 0))
    return pl.pallas_call(
        _src_embed_body,
        grid=grid,
        in_specs=[
            pl.BlockSpec((rb, IN_CH), lambda i: (i, 0)),
            pl.BlockSpec((rb, 4), lambda i: (i, 0)),
            full((IN_CH, HID)), full((4, HID)), full((1, HID)),
            full((HID, HID)), full((1, HID)), full((1, HID)), full((1, HID)),
            full((HID, HID)), full((HID, HID)),
        ],
        out_specs=[pl.BlockSpec((rb, HID), lambda i: (i, 0))] * 3,
        out_shape=[jax.ShapeDtypeStruct((n, HID), jnp.float32)] * 3,
        interpret=_INTERPRET,
    )(x, ll, p['w1'][:IN_CH], p['w1'][IN_CH:], p['b1'].reshape(1, -1),
      p['w2'], p['b2'].reshape(1, -1), p['g'].reshape(1, -1),
      p['bln'].reshape(1, -1), a_w, bdec_w)


def _dst_embed_body(ll, w1, b1, w2, b2, g, bln, benc_w, xd_out, pd_out):
    h = _silu(_dot(ll[...], w1[...]) + b1[...])
    xd = _ln(_dot(h, w2[...]) + b2[...], g[...], bln[...])
    xd_out[...] = xd
    pd_out[...] = _dot(xd, benc_w[...])


def _dst_embed(ll, p, benc_w, rb=2000):
    n = ll.shape[0]
    grid = (n // rb,)
    full = lambda shp: pl.BlockSpec(shp, lambda i: (0, 0))
    return pl.pallas_call(
        _dst_embed_body,
        grid=grid,
        in_specs=[
            pl.BlockSpec((rb, 4), lambda i: (i, 0)),
            full((4, HID)), full((1, HID)), full((HID, HID)), full((1, HID)),
            full((1, HID)), full((1, HID)), full((HID, HID)),
        ],
        out_specs=[pl.BlockSpec((rb, HID), lambda i: (i, 0))] * 2,
        out_shape=[jax.ShapeDtypeStruct((n, HID), jnp.float32)] * 2,
        interpret=_INTERPRET,
    )(ll, p['w1'], p['b1'].reshape(1, -1), p['w2'], p['b2'].reshape(1, -1),
      p['g'].reshape(1, -1), p['bln'].reshape(1, -1), benc_w)


def _edge_msg_body(sgd, e, c_w, b1, w2, b2, g, bln, m_out):
    h = _silu(sgd[...] + _dot(e[...], c_w[...]) + b1[...])
    m_out[...] = _ln(_dot(h, w2[...]) + b2[...], g[...], bln[...]) + e[...]


def _edge_msg(sgd, e, p, rb=2000):
    n = sgd.shape[0]
    grid = (n // rb,)
    full = lambda shp: pl.BlockSpec(shp, lambda i: (0, 0))
    return pl.pallas_call(
        _edge_msg_body,
        grid=grid,
        in_specs=[
            pl.BlockSpec((rb, HID), lambda i: (i, 0)),
            pl.BlockSpec((rb, HID), lambda i: (i, 0)),
            full((HID, HID)), full((1, HID)), full((HID, HID)), full((1, HID)),
            full((1, HID)), full((1, HID)),
        ],
        out_specs=pl.BlockSpec((rb, HID), lambda i: (i, 0)),
        out_shape=jax.ShapeDtypeStruct((n, HID), jnp.float32),
        interpret=_INTERPRET,
    )(sgd, e, p['w1'][2 * HID:], p['b1'].reshape(1, -1), p['w2'],
      p['b2'].reshape(1, -1), p['g'].reshape(1, -1), p['bln'].reshape(1, -1))


def _node_update_body(project, xd, agg, v1a, v1b, b1, w2, b2, g, bln, pw, pb,
                      out0, out1=None):
    h = _silu(_dot(xd[...], v1a[...]) + _dot(agg[...], v1b[...]) + b1[...])
    xn = xd[...] + _ln(_dot(h, w2[...]) + b2[...], g[...], bln[...])
    if project:
        out0[...] = _dot(xn, pw[...]) + pb[...]
    else:
        out0[...] = xn
        out1[...] = _dot(xn, pw[...]) + pb[...]


def _node_update(xd, agg, p, pw, pb, project, rb=2000):
    n = xd.shape[0]
    grid = (n // rb,)
    full = lambda shp: pl.BlockSpec(shp, lambda i: (0, 0))
    pout = pw.shape[1]
    if project:
        out_specs = pl.BlockSpec((rb, pout), lambda i: (i, 0))
        out_shape = jax.ShapeDtypeStruct((n, pout), jnp.float32)
    else:
        out_specs = [pl.BlockSpec((rb, HID), lambda i: (i, 0)),
                     pl.BlockSpec((rb, pout), lambda i: (i, 0))]
        out_shape = [jax.ShapeDtypeStruct((n, HID), jnp.float32),
                     jax.ShapeDtypeStruct((n, pout), jnp.float32)]
    return pl.pallas_call(
        functools.partial(_node_update_body, project),
        grid=grid,
        in_specs=[
            pl.BlockSpec((rb, HID), lambda i: (i, 0)),
            pl.BlockSpec((rb, HID), lambda i: (i, 0)),
            full((HID, HID)), full((HID, HID)), full((1, HID)),
            full((HID, HID)), full((1, HID)), full((1, HID)), full((1, HID)),
            full((HID, pout)), full((1, pout)),
        ],
        out_specs=out_specs,
        out_shape=out_shape,
        interpret=_INTERPRET,
    )(xd, agg, p['w1'][:HID], p['w1'][HID:], p['b1'].reshape(1, -1),
      p['w2'], p['b2'].reshape(1, -1), p['g'].reshape(1, -1),
      p['bln'].reshape(1, -1), pw, pb.reshape(1, -1))


_NC = 2
_NS = 16
_NW = _NC * _NS


def _gather_add(ps, pd, src_idx, dst_idx):
    n = src_idx.shape[0]
    ch = n // _NW
    K = 200
    nch = ch // K
    assert ch * _NW == n and nch * K == ch and K % 8 == 0 and nch % 2 == 1

    mesh = plsc.VectorSubcoreMesh(core_axis_name="c", subcore_axis_name="s")
    vm = lambda *s: pltpu.VMEM(s, jnp.float32)

    @functools.partial(
        pl.kernel, mesh=mesh,
        out_type=jax.ShapeDtypeStruct((n, HID), jnp.float32),
        scratch_types=[
            pltpu.VMEM((K,), jnp.int32), pltpu.VMEM((K,), jnp.int32),
            pltpu.VMEM((K,), jnp.int32), pltpu.VMEM((K,), jnp.int32),
            vm(K, HID), vm(K, HID), vm(K, HID), vm(K, HID),
            pltpu.SemaphoreType.DMA, pltpu.SemaphoreType.DMA,
            pltpu.SemaphoreType.DMA, pltpu.SemaphoreType.DMA,
        ],
    )
    def k(ps_hbm, pd_hbm, si_hbm, di_hbm, out_hbm, si0, si1, di0, di1,
          ra0, rb0, ra1, rb1, sa0, sb0, sa1, sb1):
        wid = lax.axis_index("s") * _NC + lax.axis_index("c")
        base0 = wid * ch
        sis, dis = (si0, si1), (di0, di1)
        ras, rbs = (ra0, ra1), (rb0, rb1)
        sas, sbs = (sa0, sa1), (sb0, sb1)

        def issue(c, b):
            base = base0 + c * K
            pltpu.sync_copy(si_hbm.at[pl.ds(base, K)], sis[b])
            pltpu.sync_copy(di_hbm.at[pl.ds(base, K)], dis[b])
            pltpu.async_copy(ps_hbm.at[sis[b]], ras[b], sas[b])
            pltpu.async_copy(pd_hbm.at[dis[b]], rbs[b], sbs[b])

        def finish(c, b):
            ra, rb = ras[b], rbs[b]
            pltpu.make_async_copy(ps_hbm.at[sis[b]], ra, sas[b]).wait()
            pltpu.make_async_copy(pd_hbm.at[dis[b]], rb, sbs[b]).wait()

            def row(r, c2):
                for j in range(HID // 16):
                    sl = pl.ds(j * 16, 16)
                    rb[r, sl] = ra[r, sl] + rb[r, sl]
                return c2
            lax.fori_loop(0, K, row, 0)
            pltpu.sync_copy(rb, out_hbm.at[pl.ds(base0 + c * K, K)])

        def chunk(i, carry):
            issue(i, 0)
            finish(i, 0)
            return carry
        lax.fori_loop(0, nch, chunk, 0)

    return k(ps, pd, src_idx, dst_idx)


def _seg_sum_kernel(n_seg, e_total):
    tile_rows = -(-n_seg // (_NW * 8)) * 8
    passes = -(-tile_rows // 784)
    sub = -(-tile_rows // (passes * 8)) * 8
    out_rows = _NW * sub * passes
    DUMP = sub
    IDC = 640
    nidc = e_total // IDC
    assert nidc * IDC == e_total
    EPAD = (-(-e_total // 1024) + 2) * 1024
    mesh = plsc.VectorSubcoreMesh(core_axis_name="c", subcore_axis_name="s")

    @functools.partial(
        pl.kernel, mesh=mesh,
        out_type=[jax.ShapeDtypeStruct((out_rows, HID), jnp.float32),
                  jax.ShapeDtypeStruct((_NW, EPAD), jnp.int32),
                  jax.ShapeDtypeStruct((_NW, EPAD), jnp.int32)],
        scratch_types=[
            pltpu.VMEM((sub + 1, HID), jnp.float32),
            pltpu.VMEM((IDC,), jnp.int32),
            pltpu.VMEM((2048,), jnp.int32),
            pltpu.VMEM((2048,), jnp.int32),
            pltpu.VMEM((1024,), jnp.int32),
            pltpu.VMEM((128, HID), jnp.float32),
            pltpu.SMEM((1024,), jnp.int32),
            pltpu.SemaphoreType.DMA,
        ],
    )
    def k(m_hbm, di_hbm, agg, bin_d, bin_e, acc, idb, cb_d, cb_e,
          blk_e, rows, sm_d, sem):
        t = lax.axis_index("s") * _NC + lax.axis_index("c")
        iota = lax.iota(jnp.int32, 16)

        for p in range(passes):
            lo = t * sub * passes + p * sub
            hi = lo + sub
            def zrow(r, c_):
                for j in range(HID // 16):
                    acc[r, pl.ds(16 * j, 16)] = jnp.zeros((16,), jnp.float32)
                return c_
            lax.fori_loop(0, sub + 1, zrow, 0)

            def chunk(ci, carry):
                pos, nblk = carry
                pltpu.sync_copy(di_hbm.at[pl.ds(ci * IDC, IDC)], idb)

                def vreg(j, pos2):
                    d = idb[pl.ds(j * 16, 16)]
                    msk = (d >= lo) & (d < hi)
                    eidv = iota + (ci * IDC + j * 16)
                    inc = jnp.cumsum(msk.astype(jnp.int32))
                    idxv = pos2 + inc - 1
                    plsc.store_scatter(cb_d, [idxv], d, mask=msk)
                    plsc.store_scatter(cb_e, [idxv], eidv, mask=msk)
                    return pos2 + jnp.max(inc)
                pos = lax.fori_loop(0, IDC // 16, vreg, pos)

                @pl.when(pos >= 1024)
                def _drain():
                    pltpu.sync_copy(cb_d.at[pl.ds(0, 1024)],
                                    bin_d.at[t, pl.ds(nblk * 1024, 1024)])
                    pltpu.sync_copy(cb_e.at[pl.ds(0, 1024)],
                                    bin_e.at[t, pl.ds(nblk * 1024, 1024)])
                    for v in range(64):
                        s_, dsl = pl.ds(1024 + 16 * v, 16), pl.ds(16 * v, 16)
                        cb_d[dsl] = cb_d[s_]
                        cb_e[dsl] = cb_e[s_]
                drained = (pos >= 1024).astype(jnp.int32)
                return pos - 1024 * drained, nblk + drained
            pos, nblk = lax.fori_loop(0, nidc, chunk, (0, 0))

            pltpu.sync_copy(cb_d.at[pl.ds(0, 1024)],
                            bin_d.at[t, pl.ds(nblk * 1024, 1024)])
            pltpu.sync_copy(cb_e.at[pl.ds(0, 1024)],
                            bin_e.at[t, pl.ds(nblk * 1024, 1024)])
            total = nblk * 1024 + pos
            nblk_tot = nblk + (pos > 0).astype(jnp.int32)

            def blk(b, c_):
                pltpu.sync_copy(bin_d.at[t, pl.ds(b * 1024, 1024)], sm_d)
                pltpu.sync_copy(bin_e.at[t, pl.ds(b * 1024, 1024)], blk_e)
                for v in range(64):
                    sl = pl.ds(16 * v, 16)
                    gi = iota + (b * 1024 + 16 * v)
                    blk_e[sl] = jnp.where(gi < total, blk_e[sl], 0)

                def subblk(s_, c2):
                    pltpu.async_copy(
                        m_hbm.at[blk_e.at[pl.ds(128 * s_, 128)]], rows,
                        sem).wait()

                    def row(r, c3):
                        d = sm_d[128 * s_ + r]
                        gi = b * 1024 + 128 * s_ + r
                        ok = (d >= lo) & (d < hi) & (gi < total)
                        off = jnp.where(ok, d - lo, DUMP)
                        for j in range(HID // 16):
                            sl = pl.ds(16 * j, 16)
                            plsc.addupdate(acc.at[off, sl], rows[r, sl])
                        return c3
                    return lax.fori_loop(0, 128, row, c2)
                return lax.fori_loop(0, 8, subblk, c_)
            lax.fori_loop(0, nblk_tot, blk, 0)

            pltpu.sync_copy(acc.at[pl.ds(0, sub)], agg.at[pl.ds(lo, sub)])
    return k


@functools.lru_cache(maxsize=None)
def _seg_sum_fn(n_seg, e_total):
    return _seg_sum_kernel(n_seg, e_total)


def _seg_sum(m, dst_idx, n_seg):
    return jax.ops.segment_sum(m, dst_idx, num_segments=n_seg)


def kernel(x, params, era_latlons, h_latlons, e2h_edge_attr, h2e_edge_attr,
           e2h_edge_index, h2e_edge_index):
    enc, dec = params['enc'], params['dec']
    bs = x.shape[0]
    x_flat = x.reshape(bs * N_ERA, IN_CH)

    a_enc = enc['blk0_edge']['w1'][:HID]
    b_enc = enc['blk0_edge']['w1'][HID:2 * HID]
    a_dec = dec['blk0_edge']['w1'][:HID]
    b_dec = dec['blk0_edge']['w1'][HID:2 * HID]

    e2 = _edge_embed(h2e_edge_attr, dec['emb_edges'])
    e1 = _edge_embed(e2h_edge_attr, enc['emb_edges'])
    xs, ps1, pd2 = _src_embed(x_flat, era_latlons, enc['emb_src'],
                              a_enc, b_dec)
    xd, pd1 = _dst_embed(h_latlons, enc['emb_dst'], b_enc)

    sgd1 = _gather_add(ps1, pd1, e2h_edge_index[0], e2h_edge_index[1])
    m1 = _edge_msg(sgd1, e1, enc['blk0_edge'])
    agg1 = _seg_sum(m1, e2h_edge_index[1], N_H)
    xlat, ps2 = _node_update(xd, agg1, enc['blk0_node'], a_dec,
                             jnp.zeros((HID,), jnp.float32), project=False)

    sgd2 = _gather_add(ps2, pd2, h2e_edge_index[0], h2e_edge_index[1])
    m2 = _edge_msg(sgd2, e2, dec['blk0_edge'])
    agg2 = _seg_sum(m2, h2e_edge_index[1], N_ERA)
    out = _node_update(xs, agg2, dec['blk0_node'], dec['out_w'],
                       dec['out_b'], project=True)
    return out.reshape(bs, N_ERA, IN_CH)


# speedup vs baseline: 1.1050x; 1.0767x over previous
import functools

import jax
import jax.numpy as jnp
from jax import lax
from jax.experimental import pallas as pl
from jax.experimental.pallas import tpu as pltpu
from jax.experimental.pallas import tpu_sc as plsc

N_ERA = 50000
N_H = 10000
E = 160000
IN_CH = 128
HID = 128

_INTERPRET = False


def _ln(x, g, b):
    mu = jnp.mean(x, axis=-1, keepdims=True)
    var = jnp.mean((x - mu) ** 2, axis=-1, keepdims=True)
    return (x - mu) * jax.lax.rsqrt(var + 1e-5) * g + b


def _silu(x):
    return x * jax.nn.sigmoid(x)


def _dot(a, b):
    return jnp.dot(a, b, preferred_element_type=jnp.float32)


def _edge_embed_body(attr, w1, b1, w2, b2, g, bln, e_out):
    h = _silu(_dot(attr[...], w1[...]) + b1[...])
    e_out[...] = _ln(_dot(h, w2[...]) + b2[...], g[...], bln[...])


def _edge_embed(attr, p, rb=5000):
    n = attr.shape[0]
    grid = (n // rb,)
    full = lambda shp: pl.BlockSpec(shp, lambda i: (0, 0))
    return pl.pallas_call(
        _edge_embed_body,
        grid=grid,
        in_specs=[
            pl.BlockSpec((rb, 4), lambda i: (i, 0)),
            full((4, HID)), full((1, HID)), full((HID, HID)), full((1, HID)),
            full((1, HID)), full((1, HID)),
        ],
        out_specs=pl.BlockSpec((rb, HID), lambda i: (i, 0)),
        out_shape=jax.ShapeDtypeStruct((n, HID), jnp.float32),
        interpret=_INTERPRET,
    )(attr, p['w1'], p['b1'].reshape(1, -1), p['w2'], p['b2'].reshape(1, -1),
      p['g'].reshape(1, -1), p['bln'].reshape(1, -1))


def _src_embed_body(x, ll, w1x, w1l, b1, w2, b2, g, bln, a_w, bdec_w,
                    xs_out, ps_out, pd_out):
    h = _silu(_dot(x[...], w1x[...]) + _dot(ll[...], w1l[...]) + b1[...])
    xs = _ln(_dot(h, w2[...]) + b2[...], g[...], bln[...])
    xs_out[...] = xs
    ps_out[...] = _dot(xs, a_w[...])
    pd_out[...] = _dot(xs, bdec_w[...])


def _src_embed(x, ll, p, a_w, bdec_w, rb=5000):
    n = x.shape[0]
    grid = (n // rb,)
    full = lambda shp: pl.BlockSpec(shp, lambda i: (0, 0))
    return pl.pallas_call(
        _src_embed_body,
        grid=grid,
        in_specs=[
            pl.BlockSpec((rb, IN_CH), lambda i: (i, 0)),
            pl.BlockSpec((rb, 4), lambda i: (i, 0)),
            full((IN_CH, HID)), full((4, HID)), full((1, HID)),
            full((HID, HID)), full((1, HID)), full((1, HID)), full((1, HID)),
            full((HID, HID)), full((HID, HID)),
        ],
        out_specs=[pl.BlockSpec((rb, HID), lambda i: (i, 0))] * 3,
        out_shape=[jax.ShapeDtypeStruct((n, HID), jnp.float32)] * 3,
        interpret=_INTERPRET,
    )(x, ll, p['w1'][:IN_CH], p['w1'][IN_CH:], p['b1'].reshape(1, -1),
      p['w2'], p['b2'].reshape(1, -1), p['g'].reshape(1, -1),
      p['bln'].reshape(1, -1), a_w, bdec_w)


def _dst_embed_body(ll, w1, b1, w2, b2, g, bln, benc_w, xd_out, pd_out):
    h = _silu(_dot(ll[...], w1[...]) + b1[...])
    xd = _ln(_dot(h, w2[...]) + b2[...], g[...], bln[...])
    xd_out[...] = xd
    pd_out[...] = _dot(xd, benc_w[...])


def _dst_embed(ll, p, benc_w, rb=5000):
    n = ll.shape[0]
    grid = (n // rb,)
    full = lambda shp: pl.BlockSpec(shp, lambda i: (0, 0))
    return pl.pallas_call(
        _dst_embed_body,
        grid=grid,
        in_specs=[
            pl.BlockSpec((rb, 4), lambda i: (i, 0)),
            full((4, HID)), full((1, HID)), full((HID, HID)), full((1, HID)),
            full((1, HID)), full((1, HID)), full((HID, HID)),
        ],
        out_specs=[pl.BlockSpec((rb, HID), lambda i: (i, 0))] * 2,
        out_shape=[jax.ShapeDtypeStruct((n, HID), jnp.float32)] * 2,
        interpret=_INTERPRET,
    )(ll, p['w1'], p['b1'].reshape(1, -1), p['w2'], p['b2'].reshape(1, -1),
      p['g'].reshape(1, -1), p['bln'].reshape(1, -1), benc_w)


def _edge_msg_body(sgd, e, c_w, b1, w2, b2, g, bln, m_out):
    h = _silu(sgd[...] + _dot(e[...], c_w[...]) + b1[...])
    m_out[...] = _ln(_dot(h, w2[...]) + b2[...], g[...], bln[...]) + e[...]


def _edge_msg(sgd, e, p, rb=5000):
    n = sgd.shape[0]
    grid = (n // rb,)
    full = lambda shp: pl.BlockSpec(shp, lambda i: (0, 0))
    return pl.pallas_call(
        _edge_msg_body,
        grid=grid,
        in_specs=[
            pl.BlockSpec((rb, HID), lambda i: (i, 0)),
            pl.BlockSpec((rb, HID), lambda i: (i, 0)),
            full((HID, HID)), full((1, HID)), full((HID, HID)), full((1, HID)),
            full((1, HID)), full((1, HID)),
        ],
        out_specs=pl.BlockSpec((rb, HID), lambda i: (i, 0)),
        out_shape=jax.ShapeDtypeStruct((n, HID), jnp.float32),
        interpret=_INTERPRET,
    )(sgd, e, p['w1'][2 * HID:], p['b1'].reshape(1, -1), p['w2'],
      p['b2'].reshape(1, -1), p['g'].reshape(1, -1), p['bln'].reshape(1, -1))


def _node_update_body(project, xd, agg, v1a, v1b, b1, w2, b2, g, bln, pw, pb,
                      out0, out1=None):
    h = _silu(_dot(xd[...], v1a[...]) + _dot(agg[...], v1b[...]) + b1[...])
    xn = xd[...] + _ln(_dot(h, w2[...]) + b2[...], g[...], bln[...])
    if project:
        out0[...] = _dot(xn, pw[...]) + pb[...]
    else:
        out0[...] = xn
        out1[...] = _dot(xn, pw[...]) + pb[...]


def _node_update(xd, agg, p, pw, pb, project, rb=5000):
    n = xd.shape[0]
    grid = (n // rb,)
    full = lambda shp: pl.BlockSpec(shp, lambda i: (0, 0))
    pout = pw.shape[1]
    if project:
        out_specs = pl.BlockSpec((rb, pout), lambda i: (i, 0))
        out_shape = jax.ShapeDtypeStruct((n, pout), jnp.float32)
    else:
        out_specs = [pl.BlockSpec((rb, HID), lambda i: (i, 0)),
                     pl.BlockSpec((rb, pout), lambda i: (i, 0))]
        out_shape = [jax.ShapeDtypeStruct((n, HID), jnp.float32),
                     jax.ShapeDtypeStruct((n, pout), jnp.float32)]
    return pl.pallas_call(
        functools.partial(_node_update_body, project),
        grid=grid,
        in_specs=[
            pl.BlockSpec((rb, HID), lambda i: (i, 0)),
            pl.BlockSpec((rb, HID), lambda i: (i, 0)),
            full((HID, HID)), full((HID, HID)), full((1, HID)),
            full((HID, HID)), full((1, HID)), full((1, HID)), full((1, HID)),
            full((HID, pout)), full((1, pout)),
        ],
        out_specs=out_specs,
        out_shape=out_shape,
        interpret=_INTERPRET,
    )(xd, agg, p['w1'][:HID], p['w1'][HID:], p['b1'].reshape(1, -1),
      p['w2'], p['b2'].reshape(1, -1), p['g'].reshape(1, -1),
      p['bln'].reshape(1, -1), pw, pb.reshape(1, -1))


_NC = 2
_NS = 16
_NW = _NC * _NS


def _gather_add(ps, pd, src_idx, dst_idx):
    n = src_idx.shape[0]
    ch = n // _NW
    K = 200
    nch = ch // K
    assert ch * _NW == n and nch * K == ch and K % 8 == 0 and nch % 2 == 1

    mesh = plsc.VectorSubcoreMesh(core_axis_name="c", subcore_axis_name="s")
    vm = lambda *s: pltpu.VMEM(s, jnp.float32)

    @functools.partial(
        pl.kernel, mesh=mesh,
        out_type=jax.ShapeDtypeStruct((n, HID), jnp.float32),
        scratch_types=[
            pltpu.VMEM((K,), jnp.int32), pltpu.VMEM((K,), jnp.int32),
            pltpu.VMEM((K,), jnp.int32), pltpu.VMEM((K,), jnp.int32),
            vm(K, HID), vm(K, HID), vm(K, HID), vm(K, HID),
            pltpu.SemaphoreType.DMA, pltpu.SemaphoreType.DMA,
            pltpu.SemaphoreType.DMA, pltpu.SemaphoreType.DMA,
        ],
    )
    def k(ps_hbm, pd_hbm, si_hbm, di_hbm, out_hbm, si0, si1, di0, di1,
          ra0, rb0, ra1, rb1, sa0, sb0, sa1, sb1):
        wid = lax.axis_index("s") * _NC + lax.axis_index("c")
        base0 = wid * ch
        sis, dis = (si0, si1), (di0, di1)
        ras, rbs = (ra0, ra1), (rb0, rb1)
        sas, sbs = (sa0, sa1), (sb0, sb1)

        def issue(c, b):
            base = base0 + c * K
            pltpu.sync_copy(si_hbm.at[pl.ds(base, K)], sis[b])
            pltpu.sync_copy(di_hbm.at[pl.ds(base, K)], dis[b])
            pltpu.async_copy(ps_hbm.at[sis[b]], ras[b], sas[b])
            pltpu.async_copy(pd_hbm.at[dis[b]], rbs[b], sbs[b])

        def finish(c, b):
            ra, rb = ras[b], rbs[b]
            pltpu.make_async_copy(ps_hbm.at[sis[b]], ra, sas[b]).wait()
            pltpu.make_async_copy(pd_hbm.at[dis[b]], rb, sbs[b]).wait()

            def row(r, c2):
                for j in range(HID // 16):
                    sl = pl.ds(j * 16, 16)
                    rb[r, sl] = ra[r, sl] + rb[r, sl]
                return c2
            lax.fori_loop(0, K, row, 0)
            pltpu.sync_copy(rb, out_hbm.at[pl.ds(base0 + c * K, K)])

        def chunk(i, carry):
            issue(i, 0)
            finish(i, 0)
            return carry
        lax.fori_loop(0, nch, chunk, 0)

    return k(ps, pd, src_idx, dst_idx)


def _seg_sum_kernel(n_seg, e_total):
    tile_rows = -(-n_seg // (_NW * 8)) * 8
    passes = -(-tile_rows // 784)
    sub = -(-tile_rows // (passes * 8)) * 8
    out_rows = _NW * sub * passes
    DUMP = sub
    IDC = 640
    nidc = e_total // IDC
    assert nidc * IDC == e_total
    EPAD = (-(-e_total // 1024) + 2) * 1024
    mesh = plsc.VectorSubcoreMesh(core_axis_name="c", subcore_axis_name="s")

    @functools.partial(
        pl.kernel, mesh=mesh,
        out_type=[jax.ShapeDtypeStruct((out_rows, HID), jnp.float32),
                  jax.ShapeDtypeStruct((_NW, EPAD), jnp.int32),
                  jax.ShapeDtypeStruct((_NW, EPAD), jnp.int32)],
        scratch_types=[
            pltpu.VMEM((sub + 1, HID), jnp.float32),
            pltpu.VMEM((IDC,), jnp.int32),
            pltpu.VMEM((2048,), jnp.int32),
            pltpu.VMEM((2048,), jnp.int32),
            pltpu.VMEM((1024,), jnp.int32),
            pltpu.VMEM((128, HID), jnp.float32),
            pltpu.SMEM((1024,), jnp.int32),
            pltpu.SemaphoreType.DMA,
        ],
    )
    def k(m_hbm, di_hbm, agg, bin_d, bin_e, acc, idb, cb_d, cb_e,
          blk_e, rows, sm_d, sem):
        t = lax.axis_index("s") * _NC + lax.axis_index("c")
        iota = lax.iota(jnp.int32, 16)

        for p in range(passes):
            lo = t * sub * passes + p * sub
            hi = lo + sub
            def zrow(r, c_):
                for j in range(HID // 16):
                    acc[r, pl.ds(16 * j, 16)] = jnp.zeros((16,), jnp.float32)
                return c_
            lax.fori_loop(0, sub + 1, zrow, 0)

            def chunk(ci, carry):
                pos, nblk = carry
                pltpu.sync_copy(di_hbm.at[pl.ds(ci * IDC, IDC)], idb)

                def vreg(j, pos2):
                    d = idb[pl.ds(j * 16, 16)]
                    msk = (d >= lo) & (d < hi)
                    eidv = iota + (ci * IDC + j * 16)
                    inc = jnp.cumsum(msk.astype(jnp.int32))
                    idxv = pos2 + inc - 1
                    plsc.store_scatter(cb_d, [idxv], d, mask=msk)
                    plsc.store_scatter(cb_e, [idxv], eidv, mask=msk)
                    return pos2 + jnp.max(inc)
                pos = lax.fori_loop(0, IDC // 16, vreg, pos)

                @pl.when(pos >= 1024)
                def _drain():
                    pltpu.sync_copy(cb_d.at[pl.ds(0, 1024)],
                                    bin_d.at[t, pl.ds(nblk * 1024, 1024)])
                    pltpu.sync_copy(cb_e.at[pl.ds(0, 1024)],
                                    bin_e.at[t, pl.ds(nblk * 1024, 1024)])
                    for v in range(64):
                        s_, dsl = pl.ds(1024 + 16 * v, 16), pl.ds(16 * v, 16)
                        cb_d[dsl] = cb_d[s_]
                        cb_e[dsl] = cb_e[s_]
                drained = (pos >= 1024).astype(jnp.int32)
                return pos - 1024 * drained, nblk + drained
            pos, nblk = lax.fori_loop(0, nidc, chunk, (0, 0))

            pltpu.sync_copy(cb_d.at[pl.ds(0, 1024)],
                            bin_d.at[t, pl.ds(nblk * 1024, 1024)])
            pltpu.sync_copy(cb_e.at[pl.ds(0, 1024)],
                            bin_e.at[t, pl.ds(nblk * 1024, 1024)])
            total = nblk * 1024 + pos
            nblk_tot = nblk + (pos > 0).astype(jnp.int32)

            def blk(b, c_):
                pltpu.sync_copy(bin_d.at[t, pl.ds(b * 1024, 1024)], sm_d)
                pltpu.sync_copy(bin_e.at[t, pl.ds(b * 1024, 1024)], blk_e)
                for v in range(64):
                    sl = pl.ds(16 * v, 16)
                    gi = iota + (b * 1024 + 16 * v)
                    blk_e[sl] = jnp.where(gi < total, blk_e[sl], 0)

                def subblk(s_, c2):
                    pltpu.async_copy(
                        m_hbm.at[blk_e.at[pl.ds(128 * s_, 128)]], rows,
                        sem).wait()

                    def row(r, c3):
                        d = sm_d[128 * s_ + r]
                        gi = b * 1024 + 128 * s_ + r
                        ok = (d >= lo) & (d < hi) & (gi < total)
                        off = jnp.where(ok, d - lo, DUMP)
                        for j in range(HID // 16):
                            sl = pl.ds(16 * j, 16)
                            plsc.addupdate(acc.at[off, sl], rows[r, sl])
                        return c3
                    return lax.fori_loop(0, 128, row, c2)
                return lax.fori_loop(0, 8, subblk, c_)
            lax.fori_loop(0, nblk_tot, blk, 0)

            pltpu.sync_copy(acc.at[pl.ds(0, sub)], agg.at[pl.ds(lo, sub)])
    return k


@functools.lru_cache(maxsize=None)
def _seg_sum_fn(n_seg, e_total):
    return _seg_sum_kernel(n_seg, e_total)


def _seg_sum(m, dst_idx, n_seg):
    return jax.ops.segment_sum(m, dst_idx, num_segments=n_seg)


def kernel(x, params, era_latlons, h_latlons, e2h_edge_attr, h2e_edge_attr,
           e2h_edge_index, h2e_edge_index):
    enc, dec = params['enc'], params['dec']
    bs = x.shape[0]
    x_flat = x.reshape(bs * N_ERA, IN_CH)

    a_enc = enc['blk0_edge']['w1'][:HID]
    b_enc = enc['blk0_edge']['w1'][HID:2 * HID]
    a_dec = dec['blk0_edge']['w1'][:HID]
    b_dec = dec['blk0_edge']['w1'][HID:2 * HID]

    e2 = _edge_embed(h2e_edge_attr, dec['emb_edges'])
    e1 = _edge_embed(e2h_edge_attr, enc['emb_edges'])
    xs, ps1, pd2 = _src_embed(x_flat, era_latlons, enc['emb_src'],
                              a_enc, b_dec)
    xd, pd1 = _dst_embed(h_latlons, enc['emb_dst'], b_enc)

    sgd1 = _gather_add(ps1, pd1, e2h_edge_index[0], e2h_edge_index[1])
    m1 = _edge_msg(sgd1, e1, enc['blk0_edge'])
    agg1 = _seg_sum(m1, e2h_edge_index[1], N_H)
    xlat, ps2 = _node_update(xd, agg1, enc['blk0_node'], a_dec,
                             jnp.zeros((HID,), jnp.float32), project=False)

    sgd2 = _gather_add(ps2, pd2, h2e_edge_index[0], h2e_edge_index[1])
    m2 = _edge_msg(sgd2, e2, dec['blk0_edge'])
    agg2 = _seg_sum(m2, h2e_edge_index[1], N_ERA)
    out = _node_update(xs, agg2, dec['blk0_node'], dec['out_w'],
                       dec['out_b'], project=True)
    return out.reshape(bs, N_ERA, IN_CH)


# speedup vs baseline: 1.1282x; 1.0210x over previous
import functools

import jax
import jax.numpy as jnp
from jax import lax
from jax.experimental import pallas as pl
from jax.experimental.pallas import tpu as pltpu
from jax.experimental.pallas import tpu_sc as plsc

N_ERA = 50000
N_H = 10000
E = 160000
IN_CH = 128
HID = 128

_INTERPRET = False


def _ln(x, g, b):
    mu = jnp.mean(x, axis=-1, keepdims=True)
    var = jnp.mean((x - mu) ** 2, axis=-1, keepdims=True)
    return (x - mu) * jax.lax.rsqrt(var + 1e-5) * g + b


def _silu(x):
    return x * jax.nn.sigmoid(x)


def _dot(a, b):
    return jnp.dot(a, b, preferred_element_type=jnp.float32)


def _edge_embed_body(attr, w1, b1, w2, b2, g, bln, e_out):
    h = _silu(_dot(attr[...], w1[...]) + b1[...])
    e_out[...] = _ln(_dot(h, w2[...]) + b2[...], g[...], bln[...])


def _edge_embed(attr, p, rb=10000):
    n = attr.shape[0]
    grid = (n // rb,)
    full = lambda shp: pl.BlockSpec(shp, lambda i: (0, 0))
    return pl.pallas_call(
        _edge_embed_body,
        grid=grid,
        in_specs=[
            pl.BlockSpec((rb, 4), lambda i: (i, 0)),
            full((4, HID)), full((1, HID)), full((HID, HID)), full((1, HID)),
            full((1, HID)), full((1, HID)),
        ],
        out_specs=pl.BlockSpec((rb, HID), lambda i: (i, 0)),
        out_shape=jax.ShapeDtypeStruct((n, HID), jnp.float32),
        interpret=_INTERPRET,
    )(attr, p['w1'], p['b1'].reshape(1, -1), p['w2'], p['b2'].reshape(1, -1),
      p['g'].reshape(1, -1), p['bln'].reshape(1, -1))


def _src_embed_body(x, ll, w1x, w1l, b1, w2, b2, g, bln, a_w, bdec_w,
                    xs_out, ps_out, pd_out):
    h = _silu(_dot(x[...], w1x[...]) + _dot(ll[...], w1l[...]) + b1[...])
    xs = _ln(_dot(h, w2[...]) + b2[...], g[...], bln[...])
    xs_out[...] = xs
    ps_out[...] = _dot(xs, a_w[...])
    pd_out[...] = _dot(xs, bdec_w[...])


def _src_embed(x, ll, p, a_w, bdec_w, rb=10000):
    n = x.shape[0]
    grid = (n // rb,)
    full = lambda shp: pl.BlockSpec(shp, lambda i: (0, 0))
    return pl.pallas_call(
        _src_embed_body,
        grid=grid,
        in_specs=[
            pl.BlockSpec((rb, IN_CH), lambda i: (i, 0)),
            pl.BlockSpec((rb, 4), lambda i: (i, 0)),
            full((IN_CH, HID)), full((4, HID)), full((1, HID)),
            full((HID, HID)), full((1, HID)), full((1, HID)), full((1, HID)),
            full((HID, HID)), full((HID, HID)),
        ],
        out_specs=[pl.BlockSpec((rb, HID), lambda i: (i, 0))] * 3,
        out_shape=[jax.ShapeDtypeStruct((n, HID), jnp.float32)] * 3,
        interpret=_INTERPRET,
    )(x, ll, p['w1'][:IN_CH], p['w1'][IN_CH:], p['b1'].reshape(1, -1),
      p['w2'], p['b2'].reshape(1, -1), p['g'].reshape(1, -1),
      p['bln'].reshape(1, -1), a_w, bdec_w)


def _dst_embed_body(ll, w1, b1, w2, b2, g, bln, benc_w, xd_out, pd_out):
    h = _silu(_dot(ll[...], w1[...]) + b1[...])
    xd = _ln(_dot(h, w2[...]) + b2[...], g[...], bln[...])
    xd_out[...] = xd
    pd_out[...] = _dot(xd, benc_w[...])


def _dst_embed(ll, p, benc_w, rb=10000):
    n = ll.shape[0]
    grid = (n // rb,)
    full = lambda shp: pl.BlockSpec(shp, lambda i: (0, 0))
    return pl.pallas_call(
        _dst_embed_body,
        grid=grid,
        in_specs=[
            pl.BlockSpec((rb, 4), lambda i: (i, 0)),
            full((4, HID)), full((1, HID)), full((HID, HID)), full((1, HID)),
            full((1, HID)), full((1, HID)), full((HID, HID)),
        ],
        out_specs=[pl.BlockSpec((rb, HID), lambda i: (i, 0))] * 2,
        out_shape=[jax.ShapeDtypeStruct((n, HID), jnp.float32)] * 2,
        interpret=_INTERPRET,
    )(ll, p['w1'], p['b1'].reshape(1, -1), p['w2'], p['b2'].reshape(1, -1),
      p['g'].reshape(1, -1), p['bln'].reshape(1, -1), benc_w)


def _edge_msg_body(sgd, e, c_w, b1, w2, b2, g, bln, m_out):
    h = _silu(sgd[...] + _dot(e[...], c_w[...]) + b1[...])
    m_out[...] = _ln(_dot(h, w2[...]) + b2[...], g[...], bln[...]) + e[...]


def _edge_msg(sgd, e, p, rb=10000):
    n = sgd.shape[0]
    grid = (n // rb,)
    full = lambda shp: pl.BlockSpec(shp, lambda i: (0, 0))
    return pl.pallas_call(
        _edge_msg_body,
        grid=grid,
        in_specs=[
            pl.BlockSpec((rb, HID), lambda i: (i, 0)),
            pl.BlockSpec((rb, HID), lambda i: (i, 0)),
            full((HID, HID)), full((1, HID)), full((HID, HID)), full((1, HID)),
            full((1, HID)), full((1, HID)),
        ],
        out_specs=pl.BlockSpec((rb, HID), lambda i: (i, 0)),
        out_shape=jax.ShapeDtypeStruct((n, HID), jnp.float32),
        interpret=_INTERPRET,
    )(sgd, e, p['w1'][2 * HID:], p['b1'].reshape(1, -1), p['w2'],
      p['b2'].reshape(1, -1), p['g'].reshape(1, -1), p['bln'].reshape(1, -1))


def _node_update_body(project, xd, agg, v1a, v1b, b1, w2, b2, g, bln, pw, pb,
                      out0, out1=None):
    h = _silu(_dot(xd[...], v1a[...]) + _dot(agg[...], v1b[...]) + b1[...])
    xn = xd[...] + _ln(_dot(h, w2[...]) + b2[...], g[...], bln[...])
    if project:
        out0[...] = _dot(xn, pw[...]) + pb[...]
    else:
        out0[...] = xn
        out1[...] = _dot(xn, pw[...]) + pb[...]


def _node_update(xd, agg, p, pw, pb, project, rb=10000):
    n = xd.shape[0]
    grid = (n // rb,)
    full = lambda shp: pl.BlockSpec(shp, lambda i: (0, 0))
    pout = pw.shape[1]
    if project:
        out_specs = pl.BlockSpec((rb, pout), lambda i: (i, 0))
        out_shape = jax.ShapeDtypeStruct((n, pout), jnp.float32)
    else:
        out_specs = [pl.BlockSpec((rb, HID), lambda i: (i, 0)),
                     pl.BlockSpec((rb, pout), lambda i: (i, 0))]
        out_shape = [jax.ShapeDtypeStruct((n, HID), jnp.float32),
                     jax.ShapeDtypeStruct((n, pout), jnp.float32)]
    return pl.pallas_call(
        functools.partial(_node_update_body, project),
        grid=grid,
        in_specs=[
            pl.BlockSpec((rb, HID), lambda i: (i, 0)),
            pl.BlockSpec((rb, HID), lambda i: (i, 0)),
            full((HID, HID)), full((HID, HID)), full((1, HID)),
            full((HID, HID)), full((1, HID)), full((1, HID)), full((1, HID)),
            full((HID, pout)), full((1, pout)),
        ],
        out_specs=out_specs,
        out_shape=out_shape,
        interpret=_INTERPRET,
    )(xd, agg, p['w1'][:HID], p['w1'][HID:], p['b1'].reshape(1, -1),
      p['w2'], p['b2'].reshape(1, -1), p['g'].reshape(1, -1),
      p['bln'].reshape(1, -1), pw, pb.reshape(1, -1))


_NC = 2
_NS = 16
_NW = _NC * _NS


def _gather_add(ps, pd, src_idx, dst_idx):
    n = src_idx.shape[0]
    ch = n // _NW
    K = 200
    nch = ch // K
    assert ch * _NW == n and nch * K == ch and K % 8 == 0 and nch % 2 == 1

    mesh = plsc.VectorSubcoreMesh(core_axis_name="c", subcore_axis_name="s")
    vm = lambda *s: pltpu.VMEM(s, jnp.float32)

    @functools.partial(
        pl.kernel, mesh=mesh,
        out_type=jax.ShapeDtypeStruct((n, HID), jnp.float32),
        scratch_types=[
            pltpu.VMEM((K,), jnp.int32), pltpu.VMEM((K,), jnp.int32),
            pltpu.VMEM((K,), jnp.int32), pltpu.VMEM((K,), jnp.int32),
            vm(K, HID), vm(K, HID), vm(K, HID), vm(K, HID),
            pltpu.SemaphoreType.DMA, pltpu.SemaphoreType.DMA,
            pltpu.SemaphoreType.DMA, pltpu.SemaphoreType.DMA,
        ],
    )
    def k(ps_hbm, pd_hbm, si_hbm, di_hbm, out_hbm, si0, si1, di0, di1,
          ra0, rb0, ra1, rb1, sa0, sb0, sa1, sb1):
        wid = lax.axis_index("s") * _NC + lax.axis_index("c")
        base0 = wid * ch
        sis, dis = (si0, si1), (di0, di1)
        ras, rbs = (ra0, ra1), (rb0, rb1)
        sas, sbs = (sa0, sa1), (sb0, sb1)

        def issue(c, b):
            base = base0 + c * K
            pltpu.sync_copy(si_hbm.at[pl.ds(base, K)], sis[b])
            pltpu.sync_copy(di_hbm.at[pl.ds(base, K)], dis[b])
            pltpu.async_copy(ps_hbm.at[sis[b]], ras[b], sas[b])
            pltpu.async_copy(pd_hbm.at[dis[b]], rbs[b], sbs[b])

        def finish(c, b):
            ra, rb = ras[b], rbs[b]
            pltpu.make_async_copy(ps_hbm.at[sis[b]], ra, sas[b]).wait()
            pltpu.make_async_copy(pd_hbm.at[dis[b]], rb, sbs[b]).wait()

            def row(r, c2):
                for j in range(HID // 16):
                    sl = pl.ds(j * 16, 16)
                    rb[r, sl] = ra[r, sl] + rb[r, sl]
                return c2
            lax.fori_loop(0, K, row, 0)
            pltpu.sync_copy(rb, out_hbm.at[pl.ds(base0 + c * K, K)])

        def chunk(i, carry):
            issue(i, 0)
            finish(i, 0)
            return carry
        lax.fori_loop(0, nch, chunk, 0)

    return k(ps, pd, src_idx, dst_idx)


def _seg_sum_kernel(n_seg, e_total):
    tile_rows = -(-n_seg // (_NW * 8)) * 8
    passes = -(-tile_rows // 784)
    sub = -(-tile_rows // (passes * 8)) * 8
    out_rows = _NW * sub * passes
    DUMP = sub
    IDC = 640
    nidc = e_total // IDC
    assert nidc * IDC == e_total
    EPAD = (-(-e_total // 1024) + 2) * 1024
    mesh = plsc.VectorSubcoreMesh(core_axis_name="c", subcore_axis_name="s")

    @functools.partial(
        pl.kernel, mesh=mesh,
        out_type=[jax.ShapeDtypeStruct((out_rows, HID), jnp.float32),
                  jax.ShapeDtypeStruct((_NW, EPAD), jnp.int32),
                  jax.ShapeDtypeStruct((_NW, EPAD), jnp.int32)],
        scratch_types=[
            pltpu.VMEM((sub + 1, HID), jnp.float32),
            pltpu.VMEM((IDC,), jnp.int32),
            pltpu.VMEM((2048,), jnp.int32),
            pltpu.VMEM((2048,), jnp.int32),
            pltpu.VMEM((1024,), jnp.int32),
            pltpu.VMEM((128, HID), jnp.float32),
            pltpu.SMEM((1024,), jnp.int32),
            pltpu.SemaphoreType.DMA,
        ],
    )
    def k(m_hbm, di_hbm, agg, bin_d, bin_e, acc, idb, cb_d, cb_e,
          blk_e, rows, sm_d, sem):
        t = lax.axis_index("s") * _NC + lax.axis_index("c")
        iota = lax.iota(jnp.int32, 16)

        for p in range(passes):
            lo = t * sub * passes + p * sub
            hi = lo + sub
            def zrow(r, c_):
                for j in range(HID // 16):
                    acc[r, pl.ds(16 * j, 16)] = jnp.zeros((16,), jnp.float32)
                return c_
            lax.fori_loop(0, sub + 1, zrow, 0)

            def chunk(ci, carry):
                pos, nblk = carry
                pltpu.sync_copy(di_hbm.at[pl.ds(ci * IDC, IDC)], idb)

                def vreg(j, pos2):
                    d = idb[pl.ds(j * 16, 16)]
                    msk = (d >= lo) & (d < hi)
                    eidv = iota + (ci * IDC + j * 16)
                    inc = jnp.cumsum(msk.astype(jnp.int32))
                    idxv = pos2 + inc - 1
                    plsc.store_scatter(cb_d, [idxv], d, mask=msk)
                    plsc.store_scatter(cb_e, [idxv], eidv, mask=msk)
                    return pos2 + jnp.max(inc)
                pos = lax.fori_loop(0, IDC // 16, vreg, pos)

                @pl.when(pos >= 1024)
                def _drain():
                    pltpu.sync_copy(cb_d.at[pl.ds(0, 1024)],
                                    bin_d.at[t, pl.ds(nblk * 1024, 1024)])
                    pltpu.sync_copy(cb_e.at[pl.ds(0, 1024)],
                                    bin_e.at[t, pl.ds(nblk * 1024, 1024)])
                    for v in range(64):
                        s_, dsl = pl.ds(1024 + 16 * v, 16), pl.ds(16 * v, 16)
                        cb_d[dsl] = cb_d[s_]
                        cb_e[dsl] = cb_e[s_]
                drained = (pos >= 1024).astype(jnp.int32)
                return pos - 1024 * drained, nblk + drained
            pos, nblk = lax.fori_loop(0, nidc, chunk, (0, 0))

            pltpu.sync_copy(cb_d.at[pl.ds(0, 1024)],
                            bin_d.at[t, pl.ds(nblk * 1024, 1024)])
            pltpu.sync_copy(cb_e.at[pl.ds(0, 1024)],
                            bin_e.at[t, pl.ds(nblk * 1024, 1024)])
            total = nblk * 1024 + pos
            nblk_tot = nblk + (pos > 0).astype(jnp.int32)

            def blk(b, c_):
                pltpu.sync_copy(bin_d.at[t, pl.ds(b * 1024, 1024)], sm_d)
                pltpu.sync_copy(bin_e.at[t, pl.ds(b * 1024, 1024)], blk_e)
                for v in range(64):
                    sl = pl.ds(16 * v, 16)
                    gi = iota + (b * 1024 + 16 * v)
                    blk_e[sl] = jnp.where(gi < total, blk_e[sl], 0)

                def subblk(s_, c2):
                    pltpu.async_copy(
                        m_hbm.at[blk_e.at[pl.ds(128 * s_, 128)]], rows,
                        sem).wait()

                    def row(r, c3):
                        d = sm_d[128 * s_ + r]
                        gi = b * 1024 + 128 * s_ + r
                        ok = (d >= lo) & (d < hi) & (gi < total)
                        off = jnp.where(ok, d - lo, DUMP)
                        for j in range(HID // 16):
                            sl = pl.ds(16 * j, 16)
                            plsc.addupdate(acc.at[off, sl], rows[r, sl])
                        return c3
                    return lax.fori_loop(0, 128, row, c2)
                return lax.fori_loop(0, 8, subblk, c_)
            lax.fori_loop(0, nblk_tot, blk, 0)

            pltpu.sync_copy(acc.at[pl.ds(0, sub)], agg.at[pl.ds(lo, sub)])
    return k


@functools.lru_cache(maxsize=None)
def _seg_sum_fn(n_seg, e_total):
    return _seg_sum_kernel(n_seg, e_total)


def _seg_sum(m, dst_idx, n_seg):
    return jax.ops.segment_sum(m, dst_idx, num_segments=n_seg)


def kernel(x, params, era_latlons, h_latlons, e2h_edge_attr, h2e_edge_attr,
           e2h_edge_index, h2e_edge_index):
    enc, dec = params['enc'], params['dec']
    bs = x.shape[0]
    x_flat = x.reshape(bs * N_ERA, IN_CH)

    a_enc = enc['blk0_edge']['w1'][:HID]
    b_enc = enc['blk0_edge']['w1'][HID:2 * HID]
    a_dec = dec['blk0_edge']['w1'][:HID]
    b_dec = dec['blk0_edge']['w1'][HID:2 * HID]

    e2 = _edge_embed(h2e_edge_attr, dec['emb_edges'])
    e1 = _edge_embed(e2h_edge_attr, enc['emb_edges'])
    xs, ps1, pd2 = _src_embed(x_flat, era_latlons, enc['emb_src'],
                              a_enc, b_dec)
    xd, pd1 = _dst_embed(h_latlons, enc['emb_dst'], b_enc)

    sgd1 = _gather_add(ps1, pd1, e2h_edge_index[0], e2h_edge_index[1])
    m1 = _edge_msg(sgd1, e1, enc['blk0_edge'])
    agg1 = _seg_sum(m1, e2h_edge_index[1], N_H)
    xlat, ps2 = _node_update(xd, agg1, enc['blk0_node'], a_dec,
                             jnp.zeros((HID,), jnp.float32), project=False)

    sgd2 = _gather_add(ps2, pd2, h2e_edge_index[0], h2e_edge_index[1])
    m2 = _edge_msg(sgd2, e2, dec['blk0_edge'])
    agg2 = _seg_sum(m2, h2e_edge_index[1], N_ERA)
    out = _node_update(xs, agg2, dec['blk0_node'], dec['out_w'],
                       dec['out_b'], project=True)
    return out.reshape(bs, N_ERA, IN_CH)


# speedup vs baseline: 1.1370x; 1.0079x over previous
import functools

import jax
import jax.numpy as jnp
from jax import lax
from jax.experimental import pallas as pl
from jax.experimental.pallas import tpu as pltpu
from jax.experimental.pallas import tpu_sc as plsc

N_ERA = 50000
N_H = 10000
E = 160000
IN_CH = 128
HID = 128

_INTERPRET = False


def _ln(x, g, b):
    mu = jnp.mean(x, axis=-1, keepdims=True)
    var = jnp.mean((x - mu) ** 2, axis=-1, keepdims=True)
    return (x - mu) * jax.lax.rsqrt(var + 1e-5) * g + b


def _silu(x):
    return x * jax.nn.sigmoid(x)


def _dot(a, b):
    return jnp.dot(a, b, preferred_element_type=jnp.float32)


def _edge_embed_body(attr, w1, b1, w2, b2, g, bln, e_out):
    h = _silu(_dot(attr[...], w1[...]) + b1[...])
    e_out[...] = _ln(_dot(h, w2[...]) + b2[...], g[...], bln[...])


def _edge_embed(attr, p, rb=10000):
    n = attr.shape[0]
    grid = (n // rb,)
    full = lambda shp: pl.BlockSpec(shp, lambda i: (0, 0))
    return pl.pallas_call(
        _edge_embed_body,
        grid=grid,
        in_specs=[
            pl.BlockSpec((rb, 4), lambda i: (i, 0)),
            full((4, HID)), full((1, HID)), full((HID, HID)), full((1, HID)),
            full((1, HID)), full((1, HID)),
        ],
        out_specs=pl.BlockSpec((rb, HID), lambda i: (i, 0)),
        out_shape=jax.ShapeDtypeStruct((n, HID), jnp.float32),
        interpret=_INTERPRET,
    )(attr, p['w1'], p['b1'].reshape(1, -1), p['w2'], p['b2'].reshape(1, -1),
      p['g'].reshape(1, -1), p['bln'].reshape(1, -1))


def _src_embed_body(x, ll, w1x, w1l, b1, w2, b2, g, bln, a_w, bdec_w,
                    xs_out, ps_out, pd_out):
    h = _silu(_dot(x[...], w1x[...]) + _dot(ll[...], w1l[...]) + b1[...])
    xs = _ln(_dot(h, w2[...]) + b2[...], g[...], bln[...])
    xs_out[...] = xs
    ps_out[...] = _dot(xs, a_w[...])
    pd_out[...] = _dot(xs, bdec_w[...])


def _src_embed(x, ll, p, a_w, bdec_w, rb=10000):
    n = x.shape[0]
    grid = (n // rb,)
    full = lambda shp: pl.BlockSpec(shp, lambda i: (0, 0))
    return pl.pallas_call(
        _src_embed_body,
        grid=grid,
        in_specs=[
            pl.BlockSpec((rb, IN_CH), lambda i: (i, 0)),
            pl.BlockSpec((rb, 4), lambda i: (i, 0)),
            full((IN_CH, HID)), full((4, HID)), full((1, HID)),
            full((HID, HID)), full((1, HID)), full((1, HID)), full((1, HID)),
            full((HID, HID)), full((HID, HID)),
        ],
        out_specs=[pl.BlockSpec((rb, HID), lambda i: (i, 0))] * 3,
        out_shape=[jax.ShapeDtypeStruct((n, HID), jnp.float32)] * 3,
        interpret=_INTERPRET,
    )(x, ll, p['w1'][:IN_CH], p['w1'][IN_CH:], p['b1'].reshape(1, -1),
      p['w2'], p['b2'].reshape(1, -1), p['g'].reshape(1, -1),
      p['bln'].reshape(1, -1), a_w, bdec_w)


def _dst_embed_body(ll, w1, b1, w2, b2, g, bln, benc_w, xd_out, pd_out):
    h = _silu(_dot(ll[...], w1[...]) + b1[...])
    xd = _ln(_dot(h, w2[...]) + b2[...], g[...], bln[...])
    xd_out[...] = xd
    pd_out[...] = _dot(xd, benc_w[...])


def _dst_embed(ll, p, benc_w, rb=10000):
    n = ll.shape[0]
    grid = (n // rb,)
    full = lambda shp: pl.BlockSpec(shp, lambda i: (0, 0))
    return pl.pallas_call(
        _dst_embed_body,
        grid=grid,
        in_specs=[
            pl.BlockSpec((rb, 4), lambda i: (i, 0)),
            full((4, HID)), full((1, HID)), full((HID, HID)), full((1, HID)),
            full((1, HID)), full((1, HID)), full((HID, HID)),
        ],
        out_specs=[pl.BlockSpec((rb, HID), lambda i: (i, 0))] * 2,
        out_shape=[jax.ShapeDtypeStruct((n, HID), jnp.float32)] * 2,
        interpret=_INTERPRET,
    )(ll, p['w1'], p['b1'].reshape(1, -1), p['w2'], p['b2'].reshape(1, -1),
      p['g'].reshape(1, -1), p['bln'].reshape(1, -1), benc_w)


def _edge_msg_body(sgd, e, c_w, b1, w2, b2, g, bln, m_out):
    h = _silu(sgd[...] + _dot(e[...], c_w[...]) + b1[...])
    m_out[...] = _ln(_dot(h, w2[...]) + b2[...], g[...], bln[...]) + e[...]


def _edge_msg(sgd, e, p, rb=10000):
    n = sgd.shape[0]
    grid = (n // rb,)
    full = lambda shp: pl.BlockSpec(shp, lambda i: (0, 0))
    return pl.pallas_call(
        _edge_msg_body,
        grid=grid,
        in_specs=[
            pl.BlockSpec((rb, HID), lambda i: (i, 0)),
            pl.BlockSpec((rb, HID), lambda i: (i, 0)),
            full((HID, HID)), full((1, HID)), full((HID, HID)), full((1, HID)),
            full((1, HID)), full((1, HID)),
        ],
        out_specs=pl.BlockSpec((rb, HID), lambda i: (i, 0)),
        out_shape=jax.ShapeDtypeStruct((n, HID), jnp.float32),
        interpret=_INTERPRET,
    )(sgd, e, p['w1'][2 * HID:], p['b1'].reshape(1, -1), p['w2'],
      p['b2'].reshape(1, -1), p['g'].reshape(1, -1), p['bln'].reshape(1, -1))


def _node_update_body(project, xd, agg, v1a, v1b, b1, w2, b2, g, bln, pw, pb,
                      out0, out1=None):
    h = _silu(_dot(xd[...], v1a[...]) + _dot(agg[...], v1b[...]) + b1[...])
    xn = xd[...] + _ln(_dot(h, w2[...]) + b2[...], g[...], bln[...])
    if project:
        out0[...] = _dot(xn, pw[...]) + pb[...]
    else:
        out0[...] = xn
        out1[...] = _dot(xn, pw[...]) + pb[...]


def _node_update(xd, agg, p, pw, pb, project, rb=10000):
    n = xd.shape[0]
    grid = (n // rb,)
    full = lambda shp: pl.BlockSpec(shp, lambda i: (0, 0))
    pout = pw.shape[1]
    if project:
        out_specs = pl.BlockSpec((rb, pout), lambda i: (i, 0))
        out_shape = jax.ShapeDtypeStruct((n, pout), jnp.float32)
    else:
        out_specs = [pl.BlockSpec((rb, HID), lambda i: (i, 0)),
                     pl.BlockSpec((rb, pout), lambda i: (i, 0))]
        out_shape = [jax.ShapeDtypeStruct((n, HID), jnp.float32),
                     jax.ShapeDtypeStruct((n, pout), jnp.float32)]
    return pl.pallas_call(
        functools.partial(_node_update_body, project),
        grid=grid,
        in_specs=[
            pl.BlockSpec((rb, HID), lambda i: (i, 0)),
            pl.BlockSpec((rb, HID), lambda i: (i, 0)),
            full((HID, HID)), full((HID, HID)), full((1, HID)),
            full((HID, HID)), full((1, HID)), full((1, HID)), full((1, HID)),
            full((HID, pout)), full((1, pout)),
        ],
        out_specs=out_specs,
        out_shape=out_shape,
        interpret=_INTERPRET,
    )(xd, agg, p['w1'][:HID], p['w1'][HID:], p['b1'].reshape(1, -1),
      p['w2'], p['b2'].reshape(1, -1), p['g'].reshape(1, -1),
      p['bln'].reshape(1, -1), pw, pb.reshape(1, -1))


_NC = 2
_NS = 16
_NW = _NC * _NS


def _gather_add(ps, pd, src_idx, dst_idx):
    n = src_idx.shape[0]
    ch = n // _NW
    K = 200
    nch = ch // K
    assert ch * _NW == n and nch * K == ch and K % 8 == 0 and nch % 2 == 1

    mesh = plsc.VectorSubcoreMesh(core_axis_name="c", subcore_axis_name="s")
    vm = lambda *s: pltpu.VMEM(s, jnp.float32)

    @functools.partial(
        pl.kernel, mesh=mesh,
        out_type=jax.ShapeDtypeStruct((n, HID), jnp.float32),
        scratch_types=[
            pltpu.VMEM((K,), jnp.int32), pltpu.VMEM((K,), jnp.int32),
            pltpu.VMEM((K,), jnp.int32), pltpu.VMEM((K,), jnp.int32),
            vm(K, HID), vm(K, HID), vm(K, HID), vm(K, HID),
            pltpu.SemaphoreType.DMA, pltpu.SemaphoreType.DMA,
            pltpu.SemaphoreType.DMA, pltpu.SemaphoreType.DMA,
        ],
    )
    def k(ps_hbm, pd_hbm, si_hbm, di_hbm, out_hbm, si0, si1, di0, di1,
          ra0, rb0, ra1, rb1, sa0, sb0, sa1, sb1):
        wid = lax.axis_index("s") * _NC + lax.axis_index("c")
        base0 = wid * ch
        sis, dis = (si0, si1), (di0, di1)
        ras, rbs = (ra0, ra1), (rb0, rb1)
        sas, sbs = (sa0, sa1), (sb0, sb1)

        def issue(c, b):
            base = base0 + c * K
            pltpu.sync_copy(si_hbm.at[pl.ds(base, K)], sis[b])
            pltpu.sync_copy(di_hbm.at[pl.ds(base, K)], dis[b])
            pltpu.async_copy(ps_hbm.at[sis[b]], ras[b], sas[b])
            pltpu.async_copy(pd_hbm.at[dis[b]], rbs[b], sbs[b])

        def finish(c, b):
            ra, rb = ras[b], rbs[b]
            pltpu.make_async_copy(ps_hbm.at[sis[b]], ra, sas[b]).wait()
            pltpu.make_async_copy(pd_hbm.at[dis[b]], rb, sbs[b]).wait()

            def row(r, c2):
                for j in range(HID // 16):
                    sl = pl.ds(j * 16, 16)
                    rb[r, sl] = ra[r, sl] + rb[r, sl]
                return c2
            lax.fori_loop(0, K, row, 0)
            pltpu.sync_copy(rb, out_hbm.at[pl.ds(base0 + c * K, K)])

        def chunk(i, carry):
            issue(i, 0)
            finish(i, 0)
            return carry
        lax.fori_loop(0, nch, chunk, 0)

    return k(ps, pd, src_idx, dst_idx)


def _seg_sum_kernel(n_seg, e_total):
    tile_rows = -(-n_seg // (_NW * 8)) * 8
    passes = -(-tile_rows // 784)
    sub = -(-tile_rows // (passes * 8)) * 8
    out_rows = _NW * sub * passes
    DUMP = sub
    IDC = 640
    nidc = e_total // IDC
    assert nidc * IDC == e_total
    EPAD = (-(-e_total // 1024) + 2) * 1024
    mesh = plsc.VectorSubcoreMesh(core_axis_name="c", subcore_axis_name="s")

    @functools.partial(
        pl.kernel, mesh=mesh,
        out_type=[jax.ShapeDtypeStruct((out_rows, HID), jnp.float32),
                  jax.ShapeDtypeStruct((_NW, EPAD), jnp.int32),
                  jax.ShapeDtypeStruct((_NW, EPAD), jnp.int32)],
        scratch_types=[
            pltpu.VMEM((sub + 1, HID), jnp.float32),
            pltpu.VMEM((IDC,), jnp.int32),
            pltpu.VMEM((2048,), jnp.int32),
            pltpu.VMEM((2048,), jnp.int32),
            pltpu.VMEM((1024,), jnp.int32),
            pltpu.VMEM((128, HID), jnp.float32),
            pltpu.SMEM((1024,), jnp.int32),
            pltpu.SemaphoreType.DMA,
        ],
    )
    def k(m_hbm, di_hbm, agg, bin_d, bin_e, acc, idb, cb_d, cb_e,
          blk_e, rows, sm_d, sem):
        t = lax.axis_index("s") * _NC + lax.axis_index("c")
        iota = lax.iota(jnp.int32, 16)

        for p in range(passes):
            lo = t * sub * passes + p * sub
            hi = lo + sub
            def zrow(r, c_):
                for j in range(HID // 16):
                    acc[r, pl.ds(16 * j, 16)] = jnp.zeros((16,), jnp.float32)
                return c_
            lax.fori_loop(0, sub + 1, zrow, 0)

            def chunk(ci, carry):
                pos, nblk = carry
                pltpu.sync_copy(di_hbm.at[pl.ds(ci * IDC, IDC)], idb)

                def vreg(j, pos2):
                    d = idb[pl.ds(j * 16, 16)]
                    msk = (d >= lo) & (d < hi)
                    eidv = iota + (ci * IDC + j * 16)
                    inc = jnp.cumsum(msk.astype(jnp.int32))
                    idxv = pos2 + inc - 1
                    plsc.store_scatter(cb_d, [idxv], d, mask=msk)
                    plsc.store_scatter(cb_e, [idxv], eidv, mask=msk)
                    return pos2 + jnp.max(inc)
                pos = lax.fori_loop(0, IDC // 16, vreg, pos)

                @pl.when(pos >= 1024)
                def _drain():
                    pltpu.sync_copy(cb_d.at[pl.ds(0, 1024)],
                                    bin_d.at[t, pl.ds(nblk * 1024, 1024)])
                    pltpu.sync_copy(cb_e.at[pl.ds(0, 1024)],
                                    bin_e.at[t, pl.ds(nblk * 1024, 1024)])
                    for v in range(64):
                        s_, dsl = pl.ds(1024 + 16 * v, 16), pl.ds(16 * v, 16)
                        cb_d[dsl] = cb_d[s_]
                        cb_e[dsl] = cb_e[s_]
                drained = (pos >= 1024).astype(jnp.int32)
                return pos - 1024 * drained, nblk + drained
            pos, nblk = lax.fori_loop(0, nidc, chunk, (0, 0))

            pltpu.sync_copy(cb_d.at[pl.ds(0, 1024)],
                            bin_d.at[t, pl.ds(nblk * 1024, 1024)])
            pltpu.sync_copy(cb_e.at[pl.ds(0, 1024)],
                            bin_e.at[t, pl.ds(nblk * 1024, 1024)])
            total = nblk * 1024 + pos
            nblk_tot = nblk + (pos > 0).astype(jnp.int32)

            def blk(b, c_):
                pltpu.sync_copy(bin_d.at[t, pl.ds(b * 1024, 1024)], sm_d)
                pltpu.sync_copy(bin_e.at[t, pl.ds(b * 1024, 1024)], blk_e)
                for v in range(64):
                    sl = pl.ds(16 * v, 16)
                    gi = iota + (b * 1024 + 16 * v)
                    blk_e[sl] = jnp.where(gi < total, blk_e[sl], 0)

                def subblk(s_, c2):
                    pltpu.async_copy(
                        m_hbm.at[blk_e.at[pl.ds(128 * s_, 128)]], rows,
                        sem).wait()

                    def row(r, c3):
                        d = sm_d[128 * s_ + r]
                        gi = b * 1024 + 128 * s_ + r
                        ok = (d >= lo) & (d < hi) & (gi < total)
                        off = jnp.where(ok, d - lo, DUMP)
                        for j in range(HID // 16):
                            sl = pl.ds(16 * j, 16)
                            plsc.addupdate(acc.at[off, sl], rows[r, sl])
                        return c3
                    return lax.fori_loop(0, 128, row, c2)
                return lax.fori_loop(0, 8, subblk, c_)
            lax.fori_loop(0, nblk_tot, blk, 0)

            pltpu.sync_copy(acc.at[pl.ds(0, sub)], agg.at[pl.ds(lo, sub)])
    return k


@functools.lru_cache(maxsize=None)
def _seg_sum_fn(n_seg, e_total):
    return _seg_sum_kernel(n_seg, e_total)


def _seg_sum(m, dst_idx, n_seg):
    return jax.ops.segment_sum(m, dst_idx, num_segments=n_seg)


def kernel(x, params, era_latlons, h_latlons, e2h_edge_attr, h2e_edge_attr,
           e2h_edge_index, h2e_edge_index):
    enc, dec = params['enc'], params['dec']
    bs = x.shape[0]
    x_flat = x.reshape(bs * N_ERA, IN_CH)

    a_enc = enc['blk0_edge']['w1'][:HID]
    b_enc = enc['blk0_edge']['w1'][HID:2 * HID]
    a_dec = dec['blk0_edge']['w1'][:HID]
    b_dec = dec['blk0_edge']['w1'][HID:2 * HID]

    e2 = _edge_embed(h2e_edge_attr, dec['emb_edges'], rb=16000)
    e1 = _edge_embed(e2h_edge_attr, enc['emb_edges'], rb=16000)
    xs, ps1, pd2 = _src_embed(x_flat, era_latlons, enc['emb_src'],
                              a_enc, b_dec, rb=10000)
    xd, pd1 = _dst_embed(h_latlons, enc['emb_dst'], b_enc)

    sgd1 = _gather_add(ps1, pd1, e2h_edge_index[0], e2h_edge_index[1])
    m1 = _edge_msg(sgd1, e1, enc['blk0_edge'], rb=16000)
    agg1 = _seg_sum(m1, e2h_edge_index[1], N_H)
    xlat, ps2 = _node_update(xd, agg1, enc['blk0_node'], a_dec,
                             jnp.zeros((HID,), jnp.float32), project=False)

    sgd2 = _gather_add(ps2, pd2, h2e_edge_index[0], h2e_edge_index[1])
    m2 = _edge_msg(sgd2, e2, dec['blk0_edge'], rb=16000)
    agg2 = _seg_sum(m2, h2e_edge_index[1], N_ERA)
    out = _node_update(xs, agg2, dec['blk0_node'], dec['out_w'],
                       dec['out_b'], project=True, rb=10000)
    return out.reshape(bs, N_ERA, IN_CH)


# speedup vs baseline: 1.1661x; 1.0256x over previous
import functools

import jax
import jax.numpy as jnp
from jax import lax
from jax.experimental import pallas as pl
from jax.experimental.pallas import tpu as pltpu
from jax.experimental.pallas import tpu_sc as plsc

N_ERA = 50000
N_H = 10000
E = 160000
IN_CH = 128
HID = 128

_INTERPRET = False


def _ln(x, g, b):
    mu = jnp.mean(x, axis=-1, keepdims=True)
    var = jnp.mean((x - mu) ** 2, axis=-1, keepdims=True)
    return (x - mu) * jax.lax.rsqrt(var + 1e-5) * g + b


def _silu(x):
    return x * jax.nn.sigmoid(x)


def _dot(a, b):
    return jnp.dot(a, b, preferred_element_type=jnp.float32)


def _edge_embed_body(attr, w1, b1, w2, b2, g, bln, e_out):
    h = _silu(_dot(attr[...], w1[...]) + b1[...])
    e_out[...] = _ln(_dot(h, w2[...]) + b2[...], g[...], bln[...])


def _edge_embed(attr, p, rb=10000):
    n = attr.shape[0]
    grid = (n // rb,)
    full = lambda shp: pl.BlockSpec(shp, lambda i: (0, 0))
    return pl.pallas_call(
        _edge_embed_body,
        grid=grid,
        in_specs=[
            pl.BlockSpec((rb, 4), lambda i: (i, 0)),
            full((4, HID)), full((1, HID)), full((HID, HID)), full((1, HID)),
            full((1, HID)), full((1, HID)),
        ],
        out_specs=pl.BlockSpec((rb, HID), lambda i: (i, 0)),
        out_shape=jax.ShapeDtypeStruct((n, HID), jnp.float32),
        interpret=_INTERPRET,
    )(attr, p['w1'], p['b1'].reshape(1, -1), p['w2'], p['b2'].reshape(1, -1),
      p['g'].reshape(1, -1), p['bln'].reshape(1, -1))


def _src_embed_body(x, ll, w1x, w1l, b1, w2, b2, g, bln, a_w, bdec_w,
                    xs_out, ps_out, pd_out):
    h = _silu(_dot(x[...], w1x[...]) + _dot(ll[...], w1l[...]) + b1[...])
    xs = _ln(_dot(h, w2[...]) + b2[...], g[...], bln[...])
    xs_out[...] = xs
    ps_out[...] = _dot(xs, a_w[...])
    pd_out[...] = _dot(xs, bdec_w[...])


def _src_embed(x, ll, p, a_w, bdec_w, rb=10000):
    n = x.shape[0]
    grid = (n // rb,)
    full = lambda shp: pl.BlockSpec(shp, lambda i: (0, 0))
    return pl.pallas_call(
        _src_embed_body,
        grid=grid,
        in_specs=[
            pl.BlockSpec((rb, IN_CH), lambda i: (i, 0)),
            pl.BlockSpec((rb, 4), lambda i: (i, 0)),
            full((IN_CH, HID)), full((4, HID)), full((1, HID)),
            full((HID, HID)), full((1, HID)), full((1, HID)), full((1, HID)),
            full((HID, HID)), full((HID, HID)),
        ],
        out_specs=[pl.BlockSpec((rb, HID), lambda i: (i, 0))] * 3,
        out_shape=[jax.ShapeDtypeStruct((n, HID), jnp.float32)] * 3,
        interpret=_INTERPRET,
    )(x, ll, p['w1'][:IN_CH], p['w1'][IN_CH:], p['b1'].reshape(1, -1),
      p['w2'], p['b2'].reshape(1, -1), p['g'].reshape(1, -1),
      p['bln'].reshape(1, -1), a_w, bdec_w)


def _dst_embed_body(ll, w1, b1, w2, b2, g, bln, benc_w, xd_out, pd_out):
    h = _silu(_dot(ll[...], w1[...]) + b1[...])
    xd = _ln(_dot(h, w2[...]) + b2[...], g[...], bln[...])
    xd_out[...] = xd
    pd_out[...] = _dot(xd, benc_w[...])


def _dst_embed(ll, p, benc_w, rb=10000):
    n = ll.shape[0]
    grid = (n // rb,)
    full = lambda shp: pl.BlockSpec(shp, lambda i: (0, 0))
    return pl.pallas_call(
        _dst_embed_body,
        grid=grid,
        in_specs=[
            pl.BlockSpec((rb, 4), lambda i: (i, 0)),
            full((4, HID)), full((1, HID)), full((HID, HID)), full((1, HID)),
            full((1, HID)), full((1, HID)), full((HID, HID)),
        ],
        out_specs=[pl.BlockSpec((rb, HID), lambda i: (i, 0))] * 2,
        out_shape=[jax.ShapeDtypeStruct((n, HID), jnp.float32)] * 2,
        interpret=_INTERPRET,
    )(ll, p['w1'], p['b1'].reshape(1, -1), p['w2'], p['b2'].reshape(1, -1),
      p['g'].reshape(1, -1), p['bln'].reshape(1, -1), benc_w)


def _edge_msg_body(sgd, attr, ew1, eb1, ew2, eb2, eg, ebln,
                   c_w, b1, w2, b2, g, bln, m_out):
    eh = _silu(_dot(attr[...], ew1[...]) + eb1[...])
    e = _ln(_dot(eh, ew2[...]) + eb2[...], eg[...], ebln[...])
    h = _silu(sgd[...] + _dot(e, c_w[...]) + b1[...])
    m_out[...] = _ln(_dot(h, w2[...]) + b2[...], g[...], bln[...]) + e


def _edge_msg(sgd, attr, pe, p, rb=10000):
    n = sgd.shape[0]
    grid = (n // rb,)
    full = lambda shp: pl.BlockSpec(shp, lambda i: (0, 0))
    return pl.pallas_call(
        _edge_msg_body,
        grid=grid,
        in_specs=[
            pl.BlockSpec((rb, HID), lambda i: (i, 0)),
            pl.BlockSpec((rb, 4), lambda i: (i, 0)),
            full((4, HID)), full((1, HID)), full((HID, HID)), full((1, HID)),
            full((1, HID)), full((1, HID)),
            full((HID, HID)), full((1, HID)), full((HID, HID)), full((1, HID)),
            full((1, HID)), full((1, HID)),
        ],
        out_specs=pl.BlockSpec((rb, HID), lambda i: (i, 0)),
        out_shape=jax.ShapeDtypeStruct((n, HID), jnp.float32),
        interpret=_INTERPRET,
    )(sgd, attr, pe['w1'], pe['b1'].reshape(1, -1), pe['w2'],
      pe['b2'].reshape(1, -1), pe['g'].reshape(1, -1),
      pe['bln'].reshape(1, -1),
      p['w1'][2 * HID:], p['b1'].reshape(1, -1), p['w2'],
      p['b2'].reshape(1, -1), p['g'].reshape(1, -1), p['bln'].reshape(1, -1))


def _node_update_body(project, xd, agg, v1a, v1b, b1, w2, b2, g, bln, pw, pb,
                      out0, out1=None):
    h = _silu(_dot(xd[...], v1a[...]) + _dot(agg[...], v1b[...]) + b1[...])
    xn = xd[...] + _ln(_dot(h, w2[...]) + b2[...], g[...], bln[...])
    if project:
        out0[...] = _dot(xn, pw[...]) + pb[...]
    else:
        out0[...] = xn
        out1[...] = _dot(xn, pw[...]) + pb[...]


def _node_update(xd, agg, p, pw, pb, project, rb=10000):
    n = xd.shape[0]
    grid = (n // rb,)
    full = lambda shp: pl.BlockSpec(shp, lambda i: (0, 0))
    pout = pw.shape[1]
    if project:
        out_specs = pl.BlockSpec((rb, pout), lambda i: (i, 0))
        out_shape = jax.ShapeDtypeStruct((n, pout), jnp.float32)
    else:
        out_specs = [pl.BlockSpec((rb, HID), lambda i: (i, 0)),
                     pl.BlockSpec((rb, pout), lambda i: (i, 0))]
        out_shape = [jax.ShapeDtypeStruct((n, HID), jnp.float32),
                     jax.ShapeDtypeStruct((n, pout), jnp.float32)]
    return pl.pallas_call(
        functools.partial(_node_update_body, project),
        grid=grid,
        in_specs=[
            pl.BlockSpec((rb, HID), lambda i: (i, 0)),
            pl.BlockSpec((rb, HID), lambda i: (i, 0)),
            full((HID, HID)), full((HID, HID)), full((1, HID)),
            full((HID, HID)), full((1, HID)), full((1, HID)), full((1, HID)),
            full((HID, pout)), full((1, pout)),
        ],
        out_specs=out_specs,
        out_shape=out_shape,
        interpret=_INTERPRET,
    )(xd, agg, p['w1'][:HID], p['w1'][HID:], p['b1'].reshape(1, -1),
      p['w2'], p['b2'].reshape(1, -1), p['g'].reshape(1, -1),
      p['bln'].reshape(1, -1), pw, pb.reshape(1, -1))


_NC = 2
_NS = 16
_NW = _NC * _NS


def _gather_add(ps, pd, src_idx, dst_idx):
    n = src_idx.shape[0]
    ch = n // _NW
    K = 200
    nch = ch // K
    assert ch * _NW == n and nch * K == ch and K % 8 == 0 and nch % 2 == 1

    mesh = plsc.VectorSubcoreMesh(core_axis_name="c", subcore_axis_name="s")
    vm = lambda *s: pltpu.VMEM(s, jnp.float32)

    @functools.partial(
        pl.kernel, mesh=mesh,
        out_type=jax.ShapeDtypeStruct((n, HID), jnp.float32),
        scratch_types=[
            pltpu.VMEM((K,), jnp.int32), pltpu.VMEM((K,), jnp.int32),
            pltpu.VMEM((K,), jnp.int32), pltpu.VMEM((K,), jnp.int32),
            vm(K, HID), vm(K, HID), vm(K, HID), vm(K, HID),
            pltpu.SemaphoreType.DMA, pltpu.SemaphoreType.DMA,
            pltpu.SemaphoreType.DMA, pltpu.SemaphoreType.DMA,
        ],
    )
    def k(ps_hbm, pd_hbm, si_hbm, di_hbm, out_hbm, si0, si1, di0, di1,
          ra0, rb0, ra1, rb1, sa0, sb0, sa1, sb1):
        wid = lax.axis_index("s") * _NC + lax.axis_index("c")
        base0 = wid * ch
        sis, dis = (si0, si1), (di0, di1)
        ras, rbs = (ra0, ra1), (rb0, rb1)
        sas, sbs = (sa0, sa1), (sb0, sb1)

        def issue(c, b):
            base = base0 + c * K
            pltpu.sync_copy(si_hbm.at[pl.ds(base, K)], sis[b])
            pltpu.sync_copy(di_hbm.at[pl.ds(base, K)], dis[b])
            pltpu.async_copy(ps_hbm.at[sis[b]], ras[b], sas[b])
            pltpu.async_copy(pd_hbm.at[dis[b]], rbs[b], sbs[b])

        def finish(c, b):
            ra, rb = ras[b], rbs[b]
            pltpu.make_async_copy(ps_hbm.at[sis[b]], ra, sas[b]).wait()
            pltpu.make_async_copy(pd_hbm.at[dis[b]], rb, sbs[b]).wait()

            def row(r, c2):
                for j in range(HID // 16):
                    sl = pl.ds(j * 16, 16)
                    rb[r, sl] = ra[r, sl] + rb[r, sl]
                return c2
            lax.fori_loop(0, K, row, 0)
            pltpu.sync_copy(rb, out_hbm.at[pl.ds(base0 + c * K, K)])

        def chunk(i, carry):
            issue(i, 0)
            finish(i, 0)
            return carry
        lax.fori_loop(0, nch, chunk, 0)

    return k(ps, pd, src_idx, dst_idx)


def _seg_sum_kernel(n_seg, e_total):
    tile_rows = -(-n_seg // (_NW * 8)) * 8
    passes = -(-tile_rows // 784)
    sub = -(-tile_rows // (passes * 8)) * 8
    out_rows = _NW * sub * passes
    DUMP = sub
    IDC = 640
    nidc = e_total // IDC
    assert nidc * IDC == e_total
    EPAD = (-(-e_total // 1024) + 2) * 1024
    mesh = plsc.VectorSubcoreMesh(core_axis_name="c", subcore_axis_name="s")

    @functools.partial(
        pl.kernel, mesh=mesh,
        out_type=[jax.ShapeDtypeStruct((out_rows, HID), jnp.float32),
                  jax.ShapeDtypeStruct((_NW, EPAD), jnp.int32),
                  jax.ShapeDtypeStruct((_NW, EPAD), jnp.int32)],
        scratch_types=[
            pltpu.VMEM((sub + 1, HID), jnp.float32),
            pltpu.VMEM((IDC,), jnp.int32),
            pltpu.VMEM((2048,), jnp.int32),
            pltpu.VMEM((2048,), jnp.int32),
            pltpu.VMEM((1024,), jnp.int32),
            pltpu.VMEM((128, HID), jnp.float32),
            pltpu.SMEM((1024,), jnp.int32),
            pltpu.SemaphoreType.DMA,
        ],
    )
    def k(m_hbm, di_hbm, agg, bin_d, bin_e, acc, idb, cb_d, cb_e,
          blk_e, rows, sm_d, sem):
        t = lax.axis_index("s") * _NC + lax.axis_index("c")
        iota = lax.iota(jnp.int32, 16)

        for p in range(passes):
            lo = t * sub * passes + p * sub
            hi = lo + sub
            def zrow(r, c_):
                for j in range(HID // 16):
                    acc[r, pl.ds(16 * j, 16)] = jnp.zeros((16,), jnp.float32)
                return c_
            lax.fori_loop(0, sub + 1, zrow, 0)

            def chunk(ci, carry):
                pos, nblk = carry
                pltpu.sync_copy(di_hbm.at[pl.ds(ci * IDC, IDC)], idb)

                def vreg(j, pos2):
                    d = idb[pl.ds(j * 16, 16)]
                    msk = (d >= lo) & (d < hi)
                    eidv = iota + (ci * IDC + j * 16)
                    inc = jnp.cumsum(msk.astype(jnp.int32))
                    idxv = pos2 + inc - 1
                    plsc.store_scatter(cb_d, [idxv], d, mask=msk)
                    plsc.store_scatter(cb_e, [idxv], eidv, mask=msk)
                    return pos2 + jnp.max(inc)
                pos = lax.fori_loop(0, IDC // 16, vreg, pos)

                @pl.when(pos >= 1024)
                def _drain():
                    pltpu.sync_copy(cb_d.at[pl.ds(0, 1024)],
                                    bin_d.at[t, pl.ds(nblk * 1024, 1024)])
                    pltpu.sync_copy(cb_e.at[pl.ds(0, 1024)],
                                    bin_e.at[t, pl.ds(nblk * 1024, 1024)])
                    for v in range(64):
                        s_, dsl = pl.ds(1024 + 16 * v, 16), pl.ds(16 * v, 16)
                        cb_d[dsl] = cb_d[s_]
                        cb_e[dsl] = cb_e[s_]
                drained = (pos >= 1024).astype(jnp.int32)
                return pos - 1024 * drained, nblk + drained
            pos, nblk = lax.fori_loop(0, nidc, chunk, (0, 0))

            pltpu.sync_copy(cb_d.at[pl.ds(0, 1024)],
                            bin_d.at[t, pl.ds(nblk * 1024, 1024)])
            pltpu.sync_copy(cb_e.at[pl.ds(0, 1024)],
                            bin_e.at[t, pl.ds(nblk * 1024, 1024)])
            total = nblk * 1024 + pos
            nblk_tot = nblk + (pos > 0).astype(jnp.int32)

            def blk(b, c_):
                pltpu.sync_copy(bin_d.at[t, pl.ds(b * 1024, 1024)], sm_d)
                pltpu.sync_copy(bin_e.at[t, pl.ds(b * 1024, 1024)], blk_e)
                for v in range(64):
                    sl = pl.ds(16 * v, 16)
                    gi = iota + (b * 1024 + 16 * v)
                    blk_e[sl] = jnp.where(gi < total, blk_e[sl], 0)

                def subblk(s_, c2):
                    pltpu.async_copy(
                        m_hbm.at[blk_e.at[pl.ds(128 * s_, 128)]], rows,
                        sem).wait()

                    def row(r, c3):
                        d = sm_d[128 * s_ + r]
                        gi = b * 1024 + 128 * s_ + r
                        ok = (d >= lo) & (d < hi) & (gi < total)
                        off = jnp.where(ok, d - lo, DUMP)
                        for j in range(HID // 16):
                            sl = pl.ds(16 * j, 16)
                            plsc.addupdate(acc.at[off, sl], rows[r, sl])
                        return c3
                    return lax.fori_loop(0, 128, row, c2)
                return lax.fori_loop(0, 8, subblk, c_)
            lax.fori_loop(0, nblk_tot, blk, 0)

            pltpu.sync_copy(acc.at[pl.ds(0, sub)], agg.at[pl.ds(lo, sub)])
    return k


@functools.lru_cache(maxsize=None)
def _seg_sum_fn(n_seg, e_total):
    return _seg_sum_kernel(n_seg, e_total)


def _seg_sum(m, dst_idx, n_seg):
    return jax.ops.segment_sum(m, dst_idx, num_segments=n_seg)


def kernel(x, params, era_latlons, h_latlons, e2h_edge_attr, h2e_edge_attr,
           e2h_edge_index, h2e_edge_index):
    enc, dec = params['enc'], params['dec']
    bs = x.shape[0]
    x_flat = x.reshape(bs * N_ERA, IN_CH)

    a_enc = enc['blk0_edge']['w1'][:HID]
    b_enc = enc['blk0_edge']['w1'][HID:2 * HID]
    a_dec = dec['blk0_edge']['w1'][:HID]
    b_dec = dec['blk0_edge']['w1'][HID:2 * HID]

    xs, ps1, pd2 = _src_embed(x_flat, era_latlons, enc['emb_src'],
                              a_enc, b_dec, rb=10000)
    xd, pd1 = _dst_embed(h_latlons, enc['emb_dst'], b_enc)

    sgd1 = _gather_add(ps1, pd1, e2h_edge_index[0], e2h_edge_index[1])
    m1 = _edge_msg(sgd1, e2h_edge_attr, enc['emb_edges'], enc['blk0_edge'])
    agg1 = _seg_sum(m1, e2h_edge_index[1], N_H)
    xlat, ps2 = _node_update(xd, agg1, enc['blk0_node'], a_dec,
                             jnp.zeros((HID,), jnp.float32), project=False)

    sgd2 = _gather_add(ps2, pd2, h2e_edge_index[0], h2e_edge_index[1])
    m2 = _edge_msg(sgd2, h2e_edge_attr, dec['emb_edges'], dec['blk0_edge'])
    agg2 = _seg_sum(m2, h2e_edge_index[1], N_ERA)
    out = _node_update(xs, agg2, dec['blk0_node'], dec['out_w'],
                       dec['out_b'], project=True, rb=10000)
    return out.reshape(bs, N_ERA, IN_CH)


# speedup vs baseline: 1.1668x; 1.0006x over previous
import functools

import jax
import jax.numpy as jnp
from jax import lax
from jax.experimental import pallas as pl
from jax.experimental.pallas import tpu as pltpu
from jax.experimental.pallas import tpu_sc as plsc

N_ERA = 50000
N_H = 10000
E = 160000
IN_CH = 128
HID = 128

def _ln(x, g, b):
    mu = jnp.mean(x, axis=-1, keepdims=True)
    var = jnp.mean((x - mu) ** 2, axis=-1, keepdims=True)
    return (x - mu) * jax.lax.rsqrt(var + 1e-5) * g + b


def _silu(x):
    return x * jax.nn.sigmoid(x)


def _dot(a, b):
    return jnp.dot(a, b, preferred_element_type=jnp.float32)


def _src_embed_body(x, ll, w1x, w1l, b1, w2, b2, g, bln, a_w, bdec_w,
                    xs_out, ps_out, pd_out):
    h = _silu(_dot(x[...], w1x[...]) + _dot(ll[...], w1l[...]) + b1[...])
    xs = _ln(_dot(h, w2[...]) + b2[...], g[...], bln[...])
    xs_out[...] = xs
    ps_out[...] = _dot(xs, a_w[...])
    pd_out[...] = _dot(xs, bdec_w[...])


def _src_embed(x, ll, p, a_w, bdec_w, rb=10000):
    n = x.shape[0]
    grid = (n // rb,)
    full = lambda shp: pl.BlockSpec(shp, lambda i: (0, 0))
    return pl.pallas_call(
        _src_embed_body,
        grid=grid,
        in_specs=[
            pl.BlockSpec((rb, IN_CH), lambda i: (i, 0)),
            pl.BlockSpec((rb, 4), lambda i: (i, 0)),
            full((IN_CH, HID)), full((4, HID)), full((1, HID)),
            full((HID, HID)), full((1, HID)), full((1, HID)), full((1, HID)),
            full((HID, HID)), full((HID, HID)),
        ],
        out_specs=[pl.BlockSpec((rb, HID), lambda i: (i, 0))] * 3,
        out_shape=[jax.ShapeDtypeStruct((n, HID), jnp.float32)] * 3,
    )(x, ll, p['w1'][:IN_CH], p['w1'][IN_CH:], p['b1'].reshape(1, -1),
      p['w2'], p['b2'].reshape(1, -1), p['g'].reshape(1, -1),
      p['bln'].reshape(1, -1), a_w, bdec_w)


def _dst_embed_body(ll, w1, b1, w2, b2, g, bln, benc_w, xd_out, pd_out):
    h = _silu(_dot(ll[...], w1[...]) + b1[...])
    xd = _ln(_dot(h, w2[...]) + b2[...], g[...], bln[...])
    xd_out[...] = xd
    pd_out[...] = _dot(xd, benc_w[...])


def _dst_embed(ll, p, benc_w, rb=10000):
    n = ll.shape[0]
    grid = (n // rb,)
    full = lambda shp: pl.BlockSpec(shp, lambda i: (0, 0))
    return pl.pallas_call(
        _dst_embed_body,
        grid=grid,
        in_specs=[
            pl.BlockSpec((rb, 4), lambda i: (i, 0)),
            full((4, HID)), full((1, HID)), full((HID, HID)), full((1, HID)),
            full((1, HID)), full((1, HID)), full((HID, HID)),
        ],
        out_specs=[pl.BlockSpec((rb, HID), lambda i: (i, 0))] * 2,
        out_shape=[jax.ShapeDtypeStruct((n, HID), jnp.float32)] * 2,
    )(ll, p['w1'], p['b1'].reshape(1, -1), p['w2'], p['b2'].reshape(1, -1),
      p['g'].reshape(1, -1), p['bln'].reshape(1, -1), benc_w)


def _edge_msg_body(sgd, attr, ew1, eb1, ew2, eb2, eg, ebln,
                   c_w, b1, w2, b2, g, bln, m_out):
    eh = _silu(_dot(attr[...], ew1[...]) + eb1[...])
    e = _ln(_dot(eh, ew2[...]) + eb2[...], eg[...], ebln[...])
    h = _silu(sgd[...] + _dot(e, c_w[...]) + b1[...])
    m_out[...] = _ln(_dot(h, w2[...]) + b2[...], g[...], bln[...]) + e


def _edge_msg(sgd, attr, pe, p, rb=10000):
    n = sgd.shape[0]
    grid = (n // rb,)
    full = lambda shp: pl.BlockSpec(shp, lambda i: (0, 0))
    return pl.pallas_call(
        _edge_msg_body,
        grid=grid,
        in_specs=[
            pl.BlockSpec((rb, HID), lambda i: (i, 0)),
            pl.BlockSpec((rb, 4), lambda i: (i, 0)),
            full((4, HID)), full((1, HID)), full((HID, HID)), full((1, HID)),
            full((1, HID)), full((1, HID)),
            full((HID, HID)), full((1, HID)), full((HID, HID)), full((1, HID)),
            full((1, HID)), full((1, HID)),
        ],
        out_specs=pl.BlockSpec((rb, HID), lambda i: (i, 0)),
        out_shape=jax.ShapeDtypeStruct((n, HID), jnp.float32),
    )(sgd, attr, pe['w1'], pe['b1'].reshape(1, -1), pe['w2'],
      pe['b2'].reshape(1, -1), pe['g'].reshape(1, -1),
      pe['bln'].reshape(1, -1),
      p['w1'][2 * HID:], p['b1'].reshape(1, -1), p['w2'],
      p['b2'].reshape(1, -1), p['g'].reshape(1, -1), p['bln'].reshape(1, -1))


def _node_update_body(project, xd, agg, v1a, v1b, b1, w2, b2, g, bln, pw, pb,
                      out0, out1=None):
    h = _silu(_dot(xd[...], v1a[...]) + _dot(agg[...], v1b[...]) + b1[...])
    xn = xd[...] + _ln(_dot(h, w2[...]) + b2[...], g[...], bln[...])
    if project:
        out0[...] = _dot(xn, pw[...]) + pb[...]
    else:
        out0[...] = xn
        out1[...] = _dot(xn, pw[...]) + pb[...]


def _node_update(xd, agg, p, pw, pb, project, rb=10000):
    n = xd.shape[0]
    grid = (n // rb,)
    full = lambda shp: pl.BlockSpec(shp, lambda i: (0, 0))
    pout = pw.shape[1]
    if project:
        out_specs = pl.BlockSpec((rb, pout), lambda i: (i, 0))
        out_shape = jax.ShapeDtypeStruct((n, pout), jnp.float32)
    else:
        out_specs = [pl.BlockSpec((rb, HID), lambda i: (i, 0)),
                     pl.BlockSpec((rb, pout), lambda i: (i, 0))]
        out_shape = [jax.ShapeDtypeStruct((n, HID), jnp.float32),
                     jax.ShapeDtypeStruct((n, pout), jnp.float32)]
    return pl.pallas_call(
        functools.partial(_node_update_body, project),
        grid=grid,
        in_specs=[
            pl.BlockSpec((rb, HID), lambda i: (i, 0)),
            pl.BlockSpec((rb, HID), lambda i: (i, 0)),
            full((HID, HID)), full((HID, HID)), full((1, HID)),
            full((HID, HID)), full((1, HID)), full((1, HID)), full((1, HID)),
            full((HID, pout)), full((1, pout)),
        ],
        out_specs=out_specs,
        out_shape=out_shape,
    )(xd, agg, p['w1'][:HID], p['w1'][HID:], p['b1'].reshape(1, -1),
      p['w2'], p['b2'].reshape(1, -1), p['g'].reshape(1, -1),
      p['bln'].reshape(1, -1), pw, pb.reshape(1, -1))


_NC = 2
_NS = 16
_NW = _NC * _NS


def _gather_add(ps, pd, src_idx, dst_idx):
    n = src_idx.shape[0]
    ch = n // _NW
    K = 200
    nch = ch // K
    assert ch * _NW == n and nch * K == ch and K % 8 == 0

    mesh = plsc.VectorSubcoreMesh(core_axis_name="c", subcore_axis_name="s")
    vm = lambda *s: pltpu.VMEM(s, jnp.float32)

    @functools.partial(
        pl.kernel, mesh=mesh,
        out_type=jax.ShapeDtypeStruct((n, HID), jnp.float32),
        scratch_types=[
            pltpu.VMEM((K,), jnp.int32), pltpu.VMEM((K,), jnp.int32),
            vm(K, HID), vm(K, HID),
            pltpu.SemaphoreType.DMA, pltpu.SemaphoreType.DMA,
        ],
    )
    def k(ps_hbm, pd_hbm, si_hbm, di_hbm, out_hbm, si, di, ra, rb, sa, sb):
        wid = lax.axis_index("s") * _NC + lax.axis_index("c")
        base0 = wid * ch

        def chunk(i, carry):
            base = base0 + i * K
            pltpu.sync_copy(si_hbm.at[pl.ds(base, K)], si)
            pltpu.sync_copy(di_hbm.at[pl.ds(base, K)], di)
            pltpu.async_copy(ps_hbm.at[si], ra, sa)
            pltpu.async_copy(pd_hbm.at[di], rb, sb)
            pltpu.make_async_copy(ps_hbm.at[si], ra, sa).wait()
            pltpu.make_async_copy(pd_hbm.at[di], rb, sb).wait()

            def row(r, c2):
                for j in range(HID // 16):
                    sl = pl.ds(j * 16, 16)
                    rb[r, sl] = ra[r, sl] + rb[r, sl]
                return c2
            lax.fori_loop(0, K, row, 0)
            pltpu.sync_copy(rb, out_hbm.at[pl.ds(base, K)])
            return carry
        lax.fori_loop(0, nch, chunk, 0)

    return k(ps, pd, src_idx, dst_idx)


def _seg_sum(m, dst_idx, n_seg):
    return jax.ops.segment_sum(m, dst_idx, num_segments=n_seg)


def kernel(x, params, era_latlons, h_latlons, e2h_edge_attr, h2e_edge_attr,
           e2h_edge_index, h2e_edge_index):
    enc, dec = params['enc'], params['dec']
    bs = x.shape[0]
    x_flat = x.reshape(bs * N_ERA, IN_CH)

    a_enc = enc['blk0_edge']['w1'][:HID]
    b_enc = enc['blk0_edge']['w1'][HID:2 * HID]
    a_dec = dec['blk0_edge']['w1'][:HID]
    b_dec = dec['blk0_edge']['w1'][HID:2 * HID]

    xs, ps1, pd2 = _src_embed(x_flat, era_latlons, enc['emb_src'],
                              a_enc, b_dec, rb=10000)
    xd, pd1 = _dst_embed(h_latlons, enc['emb_dst'], b_enc)

    sgd1 = _gather_add(ps1, pd1, e2h_edge_index[0], e2h_edge_index[1])
    m1 = _edge_msg(sgd1, e2h_edge_attr, enc['emb_edges'], enc['blk0_edge'])
    agg1 = _seg_sum(m1, e2h_edge_index[1], N_H)
    xlat, ps2 = _node_update(xd, agg1, enc['blk0_node'], a_dec,
                             jnp.zeros((HID,), jnp.float32), project=False)

    sgd2 = _gather_add(ps2, pd2, h2e_edge_index[0], h2e_edge_index[1])
    m2 = _edge_msg(sgd2, h2e_edge_attr, dec['emb_edges'], dec['blk0_edge'])
    agg2 = _seg_sum(m2, h2e_edge_index[1], N_ERA)
    out = _node_update(xs, agg2, dec['blk0_node'], dec['out_w'],
                       dec['out_b'], project=True, rb=10000)
    return out.reshape(bs, N_ERA, IN_CH)
